# Initial kernel scaffold; baseline (speedup 1.0000x reference)
import jax
import jax.numpy as jnp
from jax.experimental import pallas as pl


def kernel(x, edge_index, node_graph_ids, a, Wself0, Wneigh0, b0, Wself1, Wneigh1, b1, Wself2, Wneigh2, b2, Wc1, bc1, Wc2, bc2, Wc3, bc3):
    raise NotImplementedError("write your pallas kernel here")


# speedup vs baseline: 8.9098x; 8.9098x over previous
import functools

import jax
import jax.numpy as jnp
from jax import lax
from jax.experimental import pallas as pl
from jax.experimental.pallas import tpu as pltpu
from jax.experimental.pallas import tpu_sc as plsc

N = 10000
E = 320000
D_IN = 128
H = 32
B = 16
A_DIM = 8
R_DIM = 4
NET = 256

NC = 2
NS = 16
NW = NC * NS
CH = 128
NCHUNK = 80
EPT = NCHUNK * CH
EPAD = NW * EPT
NP = 10112
RZ = NP // NS

RB = 1000
NG = N // RB

_MESH = plsc.VectorSubcoreMesh(
    core_axis_name="c", subcore_axis_name="s", num_cores=NC, num_subcores=NS)


def _sc_edge_body(with_deg, src_hbm, dst_hbm, g_hbm, zeros_hbm, *rest):
    if with_deg:
        (zeros8_hbm, ones8_hbm, out_hbm, deg_hbm,
         acc_sp, deg_sp, src_v, dst_v, rows_v, ones_v, sems) = rest
    else:
        (out_hbm, acc_sp, src_v, dst_v, rows_v, sems) = rest
    c = lax.axis_index("c")
    s = lax.axis_index("s")
    wid = c * NS + s

    pltpu.sync_copy(zeros_hbm.at[pl.ds(s * RZ, RZ)], acc_sp.at[pl.ds(s * RZ, RZ)])
    pltpu.sync_copy(src_hbm.at[wid], src_v)
    pltpu.sync_copy(dst_hbm.at[wid], dst_v)
    if with_deg:
        pltpu.sync_copy(zeros8_hbm.at[pl.ds(s * RZ, RZ)], deg_sp.at[pl.ds(s * RZ, RZ)])
        pltpu.sync_copy(ones8_hbm, ones_v)
    plsc.subcore_barrier()

    pltpu.async_copy(g_hbm.at[src_v.at[0]], rows_v.at[0], sems.at[0])

    def outer(jj, carry):
        for b in range(2):
            i = jj * 2 + b

            @pl.when(i + 1 < NCHUNK)
            def _fire():
                pltpu.async_copy(
                    g_hbm.at[src_v.at[i + 1]], rows_v.at[1 - b], sems.at[1 - b])

            pltpu.make_async_copy(
                g_hbm.at[src_v.at[i]], rows_v.at[b], sems.at[b]).wait()
            pltpu.sync_copy(rows_v.at[b], acc_sp.at[dst_v.at[i]], add=True)
            if with_deg:
                pltpu.sync_copy(ones_v, deg_sp.at[dst_v.at[i]], add=True)
        return carry

    lax.fori_loop(0, NCHUNK // 2, outer, 0)
    plsc.subcore_barrier()

    pltpu.sync_copy(acc_sp.at[pl.ds(s * RZ, RZ)], out_hbm.at[c, pl.ds(s * RZ, RZ)])
    if with_deg:
        pltpu.sync_copy(deg_sp.at[pl.ds(s * RZ, RZ)], deg_hbm.at[c, pl.ds(s * RZ, RZ)])


_sc_scatter_deg = pl.kernel(
    functools.partial(_sc_edge_body, True),
    out_type=(
        jax.ShapeDtypeStruct((NC, NP, H), jnp.float32),
        jax.ShapeDtypeStruct((NC, NP, 8), jnp.float32),
    ),
    mesh=_MESH,
    compiler_params=pltpu.CompilerParams(use_tc_tiling_on_sc=False),
    scratch_types=[
        pltpu.VMEM_SHARED((NP, H), jnp.float32),
        pltpu.VMEM_SHARED((NP, 8), jnp.float32),
        pltpu.VMEM((NCHUNK, CH), jnp.int32),
        pltpu.VMEM((NCHUNK, CH), jnp.int32),
        pltpu.VMEM((2, CH, H), jnp.float32),
        pltpu.VMEM((CH, 8), jnp.float32),
        pltpu.SemaphoreType.DMA((2,)),
    ],
)

_sc_scatter = pl.kernel(
    functools.partial(_sc_edge_body, False),
    out_type=jax.ShapeDtypeStruct((NC, NP, H), jnp.float32),
    mesh=_MESH,
    compiler_params=pltpu.CompilerParams(use_tc_tiling_on_sc=False),
    scratch_types=[
        pltpu.VMEM_SHARED((NP, H), jnp.float32),
        pltpu.VMEM((NCHUNK, CH), jnp.int32),
        pltpu.VMEM((NCHUNK, CH), jnp.int32),
        pltpu.VMEM((2, CH, H), jnp.float32),
        pltpu.SemaphoreType.DMA((2,)),
    ],
)


def _tc1_body(x_ref, wn_ref, ws_ref, g_ref, xs_ref):
    x = x_ref[...]
    g_ref[...] = jnp.dot(x, wn_ref[...], preferred_element_type=jnp.float32)
    xs_ref[...] = jnp.dot(x, ws_ref[...], preferred_element_type=jnp.float32)


def _tc2_body(xs_ref, sp_ref, degp_ref, b_ref, wsn_ref, wnn_ref,
              hs_ref, g_ref, dinv_ref):
    ssum = sp_ref[0] + sp_ref[1]
    deg = degp_ref[0, :, 0:1] + degp_ref[1, :, 0:1]
    dinv = 1.0 / jnp.maximum(deg, 1.0)
    h = jnp.maximum(xs_ref[...] + ssum * dinv + b_ref[...], 0.0)
    hs_ref[...] = jnp.dot(h, wsn_ref[...], preferred_element_type=jnp.float32)
    g_ref[...] = jnp.dot(h, wnn_ref[...], preferred_element_type=jnp.float32)
    dinv_ref[...] = dinv


def _tc3_body(hs_ref, sp_ref, dinv_ref, b_ref, wsn_ref, wnn_ref,
              hs2_ref, g2_ref):
    ssum = sp_ref[0] + sp_ref[1]
    h = jnp.maximum(hs_ref[...] + ssum * dinv_ref[...] + b_ref[...], 0.0)
    hs2_ref[...] = jnp.dot(h, wsn_ref[...], preferred_element_type=jnp.float32)
    g2_ref[...] = jnp.dot(h, wnn_ref[...], preferred_element_type=jnp.float32)


def _tc4_body(hs_ref, sp_ref, dinv_ref, b_ref, mask_ref, a2_ref,
              wc1h_ref, wc1a_ref, bc1_ref, wc2_ref, bc2_ref, wc3_ref, bc3_ref,
              q_ref, acc_ref, cnt_ref):
    i = pl.program_id(0)
    ssum = sp_ref[0] + sp_ref[1]
    h3 = jnp.maximum(hs_ref[...] + ssum * dinv_ref[...] + b_ref[...], 0.0)
    m = mask_ref[...]
    dn = (((0,), (0,)), ((), ()))
    pacc = lax.dot_general(m, h3, dn,
                           preferred_element_type=jnp.float32)
    pcnt = lax.dot_general(m, jnp.ones((RB, 1), jnp.float32), dn,
                           preferred_element_type=jnp.float32)

    @pl.when(i == 0)
    def _init():
        acc_ref[...] = pacc
        cnt_ref[...] = pcnt

    @pl.when(i > 0)
    def _accum():
        acc_ref[...] += pacc
        cnt_ref[...] += pcnt

    @pl.when(i == NG - 1)
    def _finish():
        nf = acc_ref[...] / jnp.maximum(cnt_ref[...], 1.0)
        z = jnp.dot(nf, wc1h_ref[...], preferred_element_type=jnp.float32)
        z = z + jnp.dot(a2_ref[...], wc1a_ref[...],
                        preferred_element_type=jnp.float32) + bc1_ref[...]
        z = jnp.maximum(z, 0.0)
        z = jnp.maximum(jnp.dot(z, wc2_ref[...],
                                preferred_element_type=jnp.float32)
                        + bc2_ref[...], 0.0)
        q_ref[...] = jnp.dot(z, wc3_ref[...],
                             preferred_element_type=jnp.float32) + bc3_ref[...]


def _row_spec(cols):
    return pl.BlockSpec((RB, cols), lambda i: (i, 0))


def _full_spec(shape):
    nd = len(shape)
    return pl.BlockSpec(shape, lambda i, _nd=nd: (0,) * _nd)


def _part_spec(cols):
    return pl.BlockSpec((NC, RB, cols), lambda i: (0, i, 0))


_tc1 = pl.pallas_call(
    _tc1_body,
    grid=(NG,),
    in_specs=[_row_spec(D_IN), _full_spec((D_IN, H)), _full_spec((D_IN, H))],
    out_specs=(_row_spec(H), _row_spec(H)),
    out_shape=(jax.ShapeDtypeStruct((N, H), jnp.float32),
               jax.ShapeDtypeStruct((N, H), jnp.float32)),
    compiler_params=pltpu.CompilerParams(
        dimension_semantics=("parallel",)),
)

_tc2 = pl.pallas_call(
    _tc2_body,
    grid=(NG,),
    in_specs=[_row_spec(H), _part_spec(H), _part_spec(8), _full_spec((1, H)),
              _full_spec((H, H)), _full_spec((H, H))],
    out_specs=(_row_spec(H), _row_spec(H), _row_spec(1)),
    out_shape=(jax.ShapeDtypeStruct((N, H), jnp.float32),
               jax.ShapeDtypeStruct((N, H), jnp.float32),
               jax.ShapeDtypeStruct((N, 1), jnp.float32)),
    compiler_params=pltpu.CompilerParams(
        dimension_semantics=("parallel",)),
)

_tc3 = pl.pallas_call(
    _tc3_body,
    grid=(NG,),
    in_specs=[_row_spec(H), _part_spec(H), _row_spec(1), _full_spec((1, H)),
              _full_spec((H, H)), _full_spec((H, H))],
    out_specs=(_row_spec(H), _row_spec(H)),
    out_shape=(jax.ShapeDtypeStruct((N, H), jnp.float32),
               jax.ShapeDtypeStruct((N, H), jnp.float32)),
    compiler_params=pltpu.CompilerParams(
        dimension_semantics=("parallel",)),
)

_tc4 = pl.pallas_call(
    _tc4_body,
    grid=(NG,),
    in_specs=[_row_spec(H), _part_spec(H), _row_spec(1), _full_spec((1, H)),
              _row_spec(B), _full_spec((B, A_DIM)),
              _full_spec((H, NET)), _full_spec((A_DIM, NET)),
              _full_spec((1, NET)), _full_spec((NET, NET)),
              _full_spec((1, NET)), _full_spec((NET, R_DIM)),
              _full_spec((1, R_DIM))],
    out_specs=_full_spec((B, R_DIM)),
    out_shape=jax.ShapeDtypeStruct((B, R_DIM), jnp.float32),
    scratch_shapes=[pltpu.VMEM((B, H), jnp.float32),
                    pltpu.VMEM((B, 1), jnp.float32)],
    compiler_params=pltpu.CompilerParams(
        dimension_semantics=("arbitrary",)),
)


def kernel(x, edge_index, node_graph_ids, a,
           Wself0, Wneigh0, b0, Wself1, Wneigh1, b1, Wself2, Wneigh2, b2,
           Wc1, bc1, Wc2, bc2, Wc3, bc3):
    pad = EPAD - E
    src = jnp.concatenate([edge_index[0], jnp.zeros((pad,), jnp.int32)])
    dst = jnp.concatenate(
        [edge_index[1], (jnp.arange(pad, dtype=jnp.int32) % 16) + N])
    src = src.reshape(NW, NCHUNK, CH)
    dst = dst.reshape(NW, NCHUNK, CH)
    zeros32 = jnp.zeros((NP, H), jnp.float32)
    zeros8 = jnp.zeros((NP, 8), jnp.float32)
    ones8 = jnp.ones((CH, 8), jnp.float32)

    g0, xs = _tc1(x, Wneigh0, Wself0)
    s0p, degp = _sc_scatter_deg(src, dst, g0, zeros32, zeros8, ones8)
    h1s, g1, dinv = _tc2(xs, s0p[:, :N], degp[:, :N], b0.reshape(1, H),
                         Wself1, Wneigh1)
    s1p = _sc_scatter(src, dst, g1, zeros32)
    h2s, g2 = _tc3(h1s, s1p[:, :N], dinv, b1.reshape(1, H), Wself2, Wneigh2)
    s2p = _sc_scatter(src, dst, g2, zeros32)

    mask = (node_graph_ids[:, None]
            == jnp.arange(B, dtype=jnp.int32)[None, :]).astype(jnp.float32)
    a2 = jnp.squeeze(a, -1)
    q = _tc4(h2s, s2p[:, :N], dinv, b2.reshape(1, H), mask, a2,
             Wc1[:H], Wc1[H:], bc1.reshape(1, NET), Wc2,
             bc2.reshape(1, NET), Wc3, bc3.reshape(1, R_DIM))
    return q


# speedup vs baseline: 8.9894x; 1.0089x over previous
import functools

import jax
import jax.numpy as jnp
from jax import lax
from jax.experimental import pallas as pl
from jax.experimental.pallas import tpu as pltpu
from jax.experimental.pallas import tpu_sc as plsc

N = 10000
E = 320000
D_IN = 128
H = 32
B = 16
A_DIM = 8
R_DIM = 4
NET = 256

NC = 2
NS = 16
NW = NC * NS
CH = 128
NCHUNK = 80
EPT = NCHUNK * CH
EPAD = NW * EPT
NP = 10112
RZ = NP // NS

NBUF = 4

RB = 1000
NG = N // RB

_MESH = plsc.VectorSubcoreMesh(
    core_axis_name="c", subcore_axis_name="s", num_cores=NC, num_subcores=NS)


def _sc_edge_body(with_deg, src_hbm, dst_hbm, g_hbm, zeros_hbm, *rest):
    if with_deg:
        (zeros8_hbm, ones8_hbm, out_hbm, deg_hbm,
         acc_sp, deg_sp, src_v, dst_v, rows_v, ones_v, sems, dsem) = rest
    else:
        (out_hbm, acc_sp, src_v, dst_v, rows_v, sems) = rest
        dsem = None
    c = lax.axis_index("c")
    s = lax.axis_index("s")
    wid = c * NS + s

    pltpu.sync_copy(zeros_hbm.at[pl.ds(s * RZ, RZ)], acc_sp.at[pl.ds(s * RZ, RZ)])
    pltpu.sync_copy(src_hbm.at[wid], src_v)
    pltpu.sync_copy(dst_hbm.at[wid], dst_v)
    if with_deg:
        pltpu.sync_copy(zeros8_hbm.at[pl.ds(s * RZ, RZ)], deg_sp.at[pl.ds(s * RZ, RZ)])
        pltpu.sync_copy(ones8_hbm, ones_v)
    plsc.subcore_barrier()

    gsems, ssems = sems
    pltpu.async_copy(g_hbm.at[src_v.at[0]], rows_v.at[0], gsems.at[0])
    pltpu.async_copy(g_hbm.at[src_v.at[1]], rows_v.at[1], gsems.at[1])

    def outer(jj, carry):
        for b in range(NBUF):
            i = jj * NBUF + b
            bn = (b + 2) % NBUF

            @pl.when(jnp.logical_and(i >= 2, i + 2 < NCHUNK))
            def _drain():
                pltpu.make_async_copy(
                    rows_v.at[bn], acc_sp.at[dst_v.at[i]], ssems.at[bn]).wait()

            @pl.when(i + 2 < NCHUNK)
            def _fire():
                pltpu.async_copy(
                    g_hbm.at[src_v.at[i + 2]], rows_v.at[bn], gsems.at[bn])

            pltpu.make_async_copy(
                g_hbm.at[src_v.at[i]], rows_v.at[b], gsems.at[b]).wait()
            pltpu.async_copy(rows_v.at[b], acc_sp.at[dst_v.at[i]],
                             ssems.at[b], add=True)
            if with_deg:
                pltpu.async_copy(ones_v, deg_sp.at[dst_v.at[i]],
                                 dsem, add=True)
        return carry

    lax.fori_loop(0, NCHUNK // NBUF, outer, 0)

    for b in range(NBUF):
        pltpu.make_async_copy(
            rows_v.at[b], acc_sp.at[dst_v.at[0]], ssems.at[b]).wait()
    if with_deg:
        def drain_deg(i, carry):
            pltpu.make_async_copy(
                ones_v, deg_sp.at[dst_v.at[0]], dsem).wait()
            return carry
        lax.fori_loop(0, NCHUNK, drain_deg, 0)
    plsc.subcore_barrier()

    pltpu.sync_copy(acc_sp.at[pl.ds(s * RZ, RZ)], out_hbm.at[c, pl.ds(s * RZ, RZ)])
    if with_deg:
        pltpu.sync_copy(deg_sp.at[pl.ds(s * RZ, RZ)], deg_hbm.at[c, pl.ds(s * RZ, RZ)])


_sc_scatter_deg = pl.kernel(
    functools.partial(_sc_edge_body, True),
    out_type=(
        jax.ShapeDtypeStruct((NC, NP, H), jnp.float32),
        jax.ShapeDtypeStruct((NC, NP, 8), jnp.float32),
    ),
    mesh=_MESH,
    compiler_params=pltpu.CompilerParams(use_tc_tiling_on_sc=False),
    scratch_types=[
        pltpu.VMEM_SHARED((NP, H), jnp.float32),
        pltpu.VMEM_SHARED((NP, 8), jnp.float32),
        pltpu.VMEM((NCHUNK, CH), jnp.int32),
        pltpu.VMEM((NCHUNK, CH), jnp.int32),
        pltpu.VMEM((NBUF, CH, H), jnp.float32),
        pltpu.VMEM((CH, 8), jnp.float32),
        (pltpu.SemaphoreType.DMA((NBUF,)), pltpu.SemaphoreType.DMA((NBUF,))),
        pltpu.SemaphoreType.DMA,
    ],
)

_sc_scatter = pl.kernel(
    functools.partial(_sc_edge_body, False),
    out_type=jax.ShapeDtypeStruct((NC, NP, H), jnp.float32),
    mesh=_MESH,
    compiler_params=pltpu.CompilerParams(use_tc_tiling_on_sc=False),
    scratch_types=[
        pltpu.VMEM_SHARED((NP, H), jnp.float32),
        pltpu.VMEM((NCHUNK, CH), jnp.int32),
        pltpu.VMEM((NCHUNK, CH), jnp.int32),
        pltpu.VMEM((NBUF, CH, H), jnp.float32),
        (pltpu.SemaphoreType.DMA((NBUF,)), pltpu.SemaphoreType.DMA((NBUF,))),
    ],
)


def _tc1_body(x_ref, wn_ref, ws_ref, g_ref, xs_ref):
    x = x_ref[...]
    g_ref[...] = jnp.dot(x, wn_ref[...], preferred_element_type=jnp.float32)
    xs_ref[...] = jnp.dot(x, ws_ref[...], preferred_element_type=jnp.float32)


def _tc2_body(xs_ref, sp_ref, degp_ref, b_ref, wsn_ref, wnn_ref,
              hs_ref, g_ref, dinv_ref):
    ssum = sp_ref[0] + sp_ref[1]
    deg = degp_ref[0, :, 0:1] + degp_ref[1, :, 0:1]
    dinv = 1.0 / jnp.maximum(deg, 1.0)
    h = jnp.maximum(xs_ref[...] + ssum * dinv + b_ref[...], 0.0)
    hs_ref[...] = jnp.dot(h, wsn_ref[...], preferred_element_type=jnp.float32)
    g_ref[...] = jnp.dot(h, wnn_ref[...], preferred_element_type=jnp.float32)
    dinv_ref[...] = dinv


def _tc3_body(hs_ref, sp_ref, dinv_ref, b_ref, wsn_ref, wnn_ref,
              hs2_ref, g2_ref):
    ssum = sp_ref[0] + sp_ref[1]
    h = jnp.maximum(hs_ref[...] + ssum * dinv_ref[...] + b_ref[...], 0.0)
    hs2_ref[...] = jnp.dot(h, wsn_ref[...], preferred_element_type=jnp.float32)
    g2_ref[...] = jnp.dot(h, wnn_ref[...], preferred_element_type=jnp.float32)


def _tc4_body(hs_ref, sp_ref, dinv_ref, b_ref, mask_ref, a2_ref,
              wc1h_ref, wc1a_ref, bc1_ref, wc2_ref, bc2_ref, wc3_ref, bc3_ref,
              q_ref, acc_ref, cnt_ref):
    i = pl.program_id(0)
    ssum = sp_ref[0] + sp_ref[1]
    h3 = jnp.maximum(hs_ref[...] + ssum * dinv_ref[...] + b_ref[...], 0.0)
    m = mask_ref[...]
    dn = (((0,), (0,)), ((), ()))
    pacc = lax.dot_general(m, h3, dn,
                           preferred_element_type=jnp.float32)
    pcnt = lax.dot_general(m, jnp.ones((RB, 1), jnp.float32), dn,
                           preferred_element_type=jnp.float32)

    @pl.when(i == 0)
    def _init():
        acc_ref[...] = pacc
        cnt_ref[...] = pcnt

    @pl.when(i > 0)
    def _accum():
        acc_ref[...] += pacc
        cnt_ref[...] += pcnt

    @pl.when(i == NG - 1)
    def _finish():
        nf = acc_ref[...] / jnp.maximum(cnt_ref[...], 1.0)
        z = jnp.dot(nf, wc1h_ref[...], preferred_element_type=jnp.float32)
        z = z + jnp.dot(a2_ref[...], wc1a_ref[...],
                        preferred_element_type=jnp.float32) + bc1_ref[...]
        z = jnp.maximum(z, 0.0)
        z = jnp.maximum(jnp.dot(z, wc2_ref[...],
                                preferred_element_type=jnp.float32)
                        + bc2_ref[...], 0.0)
        q_ref[...] = jnp.dot(z, wc3_ref[...],
                             preferred_element_type=jnp.float32) + bc3_ref[...]


def _row_spec(cols):
    return pl.BlockSpec((RB, cols), lambda i: (i, 0))


def _full_spec(shape):
    nd = len(shape)
    return pl.BlockSpec(shape, lambda i, _nd=nd: (0,) * _nd)


def _part_spec(cols):
    return pl.BlockSpec((NC, RB, cols), lambda i: (0, i, 0))


_tc1 = pl.pallas_call(
    _tc1_body,
    grid=(NG,),
    in_specs=[_row_spec(D_IN), _full_spec((D_IN, H)), _full_spec((D_IN, H))],
    out_specs=(_row_spec(H), _row_spec(H)),
    out_shape=(jax.ShapeDtypeStruct((N, H), jnp.float32),
               jax.ShapeDtypeStruct((N, H), jnp.float32)),
    compiler_params=pltpu.CompilerParams(
        dimension_semantics=("parallel",)),
)

_tc2 = pl.pallas_call(
    _tc2_body,
    grid=(NG,),
    in_specs=[_row_spec(H), _part_spec(H), _part_spec(8), _full_spec((1, H)),
              _full_spec((H, H)), _full_spec((H, H))],
    out_specs=(_row_spec(H), _row_spec(H), _row_spec(1)),
    out_shape=(jax.ShapeDtypeStruct((N, H), jnp.float32),
               jax.ShapeDtypeStruct((N, H), jnp.float32),
               jax.ShapeDtypeStruct((N, 1), jnp.float32)),
    compiler_params=pltpu.CompilerParams(
        dimension_semantics=("parallel",)),
)

_tc3 = pl.pallas_call(
    _tc3_body,
    grid=(NG,),
    in_specs=[_row_spec(H), _part_spec(H), _row_spec(1), _full_spec((1, H)),
              _full_spec((H, H)), _full_spec((H, H))],
    out_specs=(_row_spec(H), _row_spec(H)),
    out_shape=(jax.ShapeDtypeStruct((N, H), jnp.float32),
               jax.ShapeDtypeStruct((N, H), jnp.float32)),
    compiler_params=pltpu.CompilerParams(
        dimension_semantics=("parallel",)),
)

_tc4 = pl.pallas_call(
    _tc4_body,
    grid=(NG,),
    in_specs=[_row_spec(H), _part_spec(H), _row_spec(1), _full_spec((1, H)),
              _row_spec(B), _full_spec((B, A_DIM)),
              _full_spec((H, NET)), _full_spec((A_DIM, NET)),
              _full_spec((1, NET)), _full_spec((NET, NET)),
              _full_spec((1, NET)), _full_spec((NET, R_DIM)),
              _full_spec((1, R_DIM))],
    out_specs=_full_spec((B, R_DIM)),
    out_shape=jax.ShapeDtypeStruct((B, R_DIM), jnp.float32),
    scratch_shapes=[pltpu.VMEM((B, H), jnp.float32),
                    pltpu.VMEM((B, 1), jnp.float32)],
    compiler_params=pltpu.CompilerParams(
        dimension_semantics=("arbitrary",)),
)


def kernel(x, edge_index, node_graph_ids, a,
           Wself0, Wneigh0, b0, Wself1, Wneigh1, b1, Wself2, Wneigh2, b2,
           Wc1, bc1, Wc2, bc2, Wc3, bc3):
    pad = EPAD - E
    src = jnp.concatenate([edge_index[0], jnp.zeros((pad,), jnp.int32)])
    dst = jnp.concatenate(
        [edge_index[1], (jnp.arange(pad, dtype=jnp.int32) % 16) + N])
    src = src.reshape(NW, NCHUNK, CH)
    dst = dst.reshape(NW, NCHUNK, CH)
    zeros32 = jnp.zeros((NP, H), jnp.float32)
    zeros8 = jnp.zeros((NP, 8), jnp.float32)
    ones8 = jnp.ones((CH, 8), jnp.float32)

    g0, xs = _tc1(x, Wneigh0, Wself0)
    s0p, degp = _sc_scatter_deg(src, dst, g0, zeros32, zeros8, ones8)
    h1s, g1, dinv = _tc2(xs, s0p[:, :N], degp[:, :N], b0.reshape(1, H),
                         Wself1, Wneigh1)
    s1p = _sc_scatter(src, dst, g1, zeros32)
    h2s, g2 = _tc3(h1s, s1p[:, :N], dinv, b1.reshape(1, H), Wself2, Wneigh2)
    s2p = _sc_scatter(src, dst, g2, zeros32)

    mask = (node_graph_ids[:, None]
            == jnp.arange(B, dtype=jnp.int32)[None, :]).astype(jnp.float32)
    a2 = jnp.squeeze(a, -1)
    q = _tc4(h2s, s2p[:, :N], dinv, b2.reshape(1, H), mask, a2,
             Wc1[:H], Wc1[H:], bc1.reshape(1, NET), Wc2,
             bc2.reshape(1, NET), Wc3, bc3.reshape(1, R_DIM))
    return q


# speedup vs baseline: 17.8705x; 1.9879x over previous
import functools

import jax
import jax.numpy as jnp
from jax import lax
from jax.experimental import pallas as pl
from jax.experimental.pallas import tpu as pltpu
from jax.experimental.pallas import tpu_sc as plsc

N = 10000
E = 320000
D_IN = 128
H = 32
B = 16
A_DIM = 8
R_DIM = 4
NET = 256

NC = 2
NS = 16
NW = NC * NS
CH = 128
NCHUNK = 80
EPT = NCHUNK * CH
EPAD = NW * EPT
NP = 10112
RZ = NP // NS

NBUF = 4
_DO_SCATTER = True
_DO_GATHER = True
_GATHER_SPMEM = True

RB = 1000
NG = N // RB

_MESH = plsc.VectorSubcoreMesh(
    core_axis_name="c", subcore_axis_name="s", num_cores=NC, num_subcores=NS)


def _sc_edge_body(with_deg, src_hbm, dst_hbm, g_hbm, zeros_hbm, *rest):
    if with_deg:
        (zeros8_hbm, ones8_hbm, out_hbm, deg_hbm,
         acc_sp, deg_sp, g_sp, src_v, dst_v, rows_v, ones_v, sems, dsem) = rest
    else:
        (out_hbm, acc_sp, g_sp, src_v, dst_v, rows_v, sems) = rest
        dsem = None
    c = lax.axis_index("c")
    s = lax.axis_index("s")
    wid = c * NS + s

    pltpu.sync_copy(zeros_hbm.at[pl.ds(s * RZ, RZ)], acc_sp.at[pl.ds(s * RZ, RZ)])
    pltpu.sync_copy(src_hbm.at[wid], src_v)
    pltpu.sync_copy(dst_hbm.at[wid], dst_v)
    if _GATHER_SPMEM:
        @pl.when(s < NS - 1)
        def _stage():
            pltpu.sync_copy(g_hbm.at[pl.ds(s * RZ, RZ)],
                            g_sp.at[pl.ds(s * RZ, RZ)])

        @pl.when(s == NS - 1)
        def _stage_last():
            pltpu.sync_copy(g_hbm.at[pl.ds((NS - 1) * RZ, N - (NS - 1) * RZ)],
                            g_sp.at[pl.ds((NS - 1) * RZ, N - (NS - 1) * RZ)])
        g_src = g_sp
    else:
        g_src = g_hbm
    if with_deg:
        pltpu.sync_copy(zeros8_hbm.at[pl.ds(s * RZ, RZ)], deg_sp.at[pl.ds(s * RZ, RZ)])
        pltpu.sync_copy(ones8_hbm, ones_v)
    plsc.subcore_barrier()

    gsems, ssems = sems
    if _DO_GATHER:
        pltpu.async_copy(g_src.at[src_v.at[0]], rows_v.at[0], gsems.at[0])
        pltpu.async_copy(g_src.at[src_v.at[1]], rows_v.at[1], gsems.at[1])

    def outer(jj, carry):
        for b in range(NBUF):
            i = jj * NBUF + b
            bn = (b + 2) % NBUF

            if _DO_SCATTER:
                @pl.when(jnp.logical_and(i >= 2, i + 2 < NCHUNK))
                def _drain():
                    pltpu.make_async_copy(
                        rows_v.at[bn], acc_sp.at[dst_v.at[i]], ssems.at[bn]).wait()

            if _DO_GATHER:
                @pl.when(i + 2 < NCHUNK)
                def _fire():
                    pltpu.async_copy(
                        g_src.at[src_v.at[i + 2]], rows_v.at[bn], gsems.at[bn])

                pltpu.make_async_copy(
                    g_src.at[src_v.at[i]], rows_v.at[b], gsems.at[b]).wait()
            if _DO_SCATTER:
                pltpu.async_copy(rows_v.at[b], acc_sp.at[dst_v.at[i]],
                                 ssems.at[b], add=True)
            if with_deg and _DO_SCATTER:
                pltpu.async_copy(ones_v, deg_sp.at[dst_v.at[i]],
                                 dsem, add=True)
        return carry

    lax.fori_loop(0, NCHUNK // NBUF, outer, 0)

    for b in range(NBUF) if _DO_SCATTER else []:
        pltpu.make_async_copy(
            rows_v.at[b], acc_sp.at[dst_v.at[0]], ssems.at[b]).wait()
    if with_deg and _DO_SCATTER:
        def drain_deg(i, carry):
            pltpu.make_async_copy(
                ones_v, deg_sp.at[dst_v.at[0]], dsem).wait()
            return carry
        lax.fori_loop(0, NCHUNK, drain_deg, 0)
    plsc.subcore_barrier()

    pltpu.sync_copy(acc_sp.at[pl.ds(s * RZ, RZ)], out_hbm.at[c, pl.ds(s * RZ, RZ)])
    if with_deg:
        pltpu.sync_copy(deg_sp.at[pl.ds(s * RZ, RZ)], deg_hbm.at[c, pl.ds(s * RZ, RZ)])


_sc_scatter_deg = pl.kernel(
    functools.partial(_sc_edge_body, True),
    out_type=(
        jax.ShapeDtypeStruct((NC, NP, H), jnp.float32),
        jax.ShapeDtypeStruct((NC, NP, 8), jnp.float32),
    ),
    mesh=_MESH,
    compiler_params=pltpu.CompilerParams(use_tc_tiling_on_sc=False),
    scratch_types=[
        pltpu.VMEM_SHARED((NP, H), jnp.float32),
        pltpu.VMEM_SHARED((NP, 8), jnp.float32),
        pltpu.VMEM_SHARED((NP, H), jnp.float32),
        pltpu.VMEM((NCHUNK, CH), jnp.int32),
        pltpu.VMEM((NCHUNK, CH), jnp.int32),
        pltpu.VMEM((NBUF, CH, H), jnp.float32),
        pltpu.VMEM((CH, 8), jnp.float32),
        (pltpu.SemaphoreType.DMA((NBUF,)), pltpu.SemaphoreType.DMA((NBUF,))),
        pltpu.SemaphoreType.DMA,
    ],
)

_sc_scatter = pl.kernel(
    functools.partial(_sc_edge_body, False),
    out_type=jax.ShapeDtypeStruct((NC, NP, H), jnp.float32),
    mesh=_MESH,
    compiler_params=pltpu.CompilerParams(use_tc_tiling_on_sc=False),
    scratch_types=[
        pltpu.VMEM_SHARED((NP, H), jnp.float32),
        pltpu.VMEM_SHARED((NP, H), jnp.float32),
        pltpu.VMEM((NCHUNK, CH), jnp.int32),
        pltpu.VMEM((NCHUNK, CH), jnp.int32),
        pltpu.VMEM((NBUF, CH, H), jnp.float32),
        (pltpu.SemaphoreType.DMA((NBUF,)), pltpu.SemaphoreType.DMA((NBUF,))),
    ],
)


def _tc1_body(x_ref, wn_ref, ws_ref, g_ref, xs_ref):
    x = x_ref[...]
    g_ref[...] = jnp.dot(x, wn_ref[...], preferred_element_type=jnp.float32)
    xs_ref[...] = jnp.dot(x, ws_ref[...], preferred_element_type=jnp.float32)


def _tc2_body(xs_ref, sp_ref, degp_ref, b_ref, wsn_ref, wnn_ref,
              hs_ref, g_ref, dinv_ref):
    ssum = sp_ref[0] + sp_ref[1]
    deg = degp_ref[0, :, 0:1] + degp_ref[1, :, 0:1]
    dinv = 1.0 / jnp.maximum(deg, 1.0)
    h = jnp.maximum(xs_ref[...] + ssum * dinv + b_ref[...], 0.0)
    hs_ref[...] = jnp.dot(h, wsn_ref[...], preferred_element_type=jnp.float32)
    g_ref[...] = jnp.dot(h, wnn_ref[...], preferred_element_type=jnp.float32)
    dinv_ref[...] = dinv


def _tc3_body(hs_ref, sp_ref, dinv_ref, b_ref, wsn_ref, wnn_ref,
              hs2_ref, g2_ref):
    ssum = sp_ref[0] + sp_ref[1]
    h = jnp.maximum(hs_ref[...] + ssum * dinv_ref[...] + b_ref[...], 0.0)
    hs2_ref[...] = jnp.dot(h, wsn_ref[...], preferred_element_type=jnp.float32)
    g2_ref[...] = jnp.dot(h, wnn_ref[...], preferred_element_type=jnp.float32)


def _tc4_body(hs_ref, sp_ref, dinv_ref, b_ref, mask_ref, a2_ref,
              wc1h_ref, wc1a_ref, bc1_ref, wc2_ref, bc2_ref, wc3_ref, bc3_ref,
              q_ref, acc_ref, cnt_ref):
    i = pl.program_id(0)
    ssum = sp_ref[0] + sp_ref[1]
    h3 = jnp.maximum(hs_ref[...] + ssum * dinv_ref[...] + b_ref[...], 0.0)
    m = mask_ref[...]
    dn = (((0,), (0,)), ((), ()))
    pacc = lax.dot_general(m, h3, dn,
                           preferred_element_type=jnp.float32)
    pcnt = lax.dot_general(m, jnp.ones((RB, 1), jnp.float32), dn,
                           preferred_element_type=jnp.float32)

    @pl.when(i == 0)
    def _init():
        acc_ref[...] = pacc
        cnt_ref[...] = pcnt

    @pl.when(i > 0)
    def _accum():
        acc_ref[...] += pacc
        cnt_ref[...] += pcnt

    @pl.when(i == NG - 1)
    def _finish():
        nf = acc_ref[...] / jnp.maximum(cnt_ref[...], 1.0)
        z = jnp.dot(nf, wc1h_ref[...], preferred_element_type=jnp.float32)
        z = z + jnp.dot(a2_ref[...], wc1a_ref[...],
                        preferred_element_type=jnp.float32) + bc1_ref[...]
        z = jnp.maximum(z, 0.0)
        z = jnp.maximum(jnp.dot(z, wc2_ref[...],
                                preferred_element_type=jnp.float32)
                        + bc2_ref[...], 0.0)
        q_ref[...] = jnp.dot(z, wc3_ref[...],
                             preferred_element_type=jnp.float32) + bc3_ref[...]


def _row_spec(cols):
    return pl.BlockSpec((RB, cols), lambda i: (i, 0))


def _full_spec(shape):
    nd = len(shape)
    return pl.BlockSpec(shape, lambda i, _nd=nd: (0,) * _nd)


def _part_spec(cols):
    return pl.BlockSpec((NC, RB, cols), lambda i: (0, i, 0))


_tc1 = pl.pallas_call(
    _tc1_body,
    grid=(NG,),
    in_specs=[_row_spec(D_IN), _full_spec((D_IN, H)), _full_spec((D_IN, H))],
    out_specs=(_row_spec(H), _row_spec(H)),
    out_shape=(jax.ShapeDtypeStruct((N, H), jnp.float32),
               jax.ShapeDtypeStruct((N, H), jnp.float32)),
    compiler_params=pltpu.CompilerParams(
        dimension_semantics=("parallel",)),
)

_tc2 = pl.pallas_call(
    _tc2_body,
    grid=(NG,),
    in_specs=[_row_spec(H), _part_spec(H), _part_spec(8), _full_spec((1, H)),
              _full_spec((H, H)), _full_spec((H, H))],
    out_specs=(_row_spec(H), _row_spec(H), _row_spec(1)),
    out_shape=(jax.ShapeDtypeStruct((N, H), jnp.float32),
               jax.ShapeDtypeStruct((N, H), jnp.float32),
               jax.ShapeDtypeStruct((N, 1), jnp.float32)),
    compiler_params=pltpu.CompilerParams(
        dimension_semantics=("parallel",)),
)

_tc3 = pl.pallas_call(
    _tc3_body,
    grid=(NG,),
    in_specs=[_row_spec(H), _part_spec(H), _row_spec(1), _full_spec((1, H)),
              _full_spec((H, H)), _full_spec((H, H))],
    out_specs=(_row_spec(H), _row_spec(H)),
    out_shape=(jax.ShapeDtypeStruct((N, H), jnp.float32),
               jax.ShapeDtypeStruct((N, H), jnp.float32)),
    compiler_params=pltpu.CompilerParams(
        dimension_semantics=("parallel",)),
)

_tc4 = pl.pallas_call(
    _tc4_body,
    grid=(NG,),
    in_specs=[_row_spec(H), _part_spec(H), _row_spec(1), _full_spec((1, H)),
              _row_spec(B), _full_spec((B, A_DIM)),
              _full_spec((H, NET)), _full_spec((A_DIM, NET)),
              _full_spec((1, NET)), _full_spec((NET, NET)),
              _full_spec((1, NET)), _full_spec((NET, R_DIM)),
              _full_spec((1, R_DIM))],
    out_specs=_full_spec((B, R_DIM)),
    out_shape=jax.ShapeDtypeStruct((B, R_DIM), jnp.float32),
    scratch_shapes=[pltpu.VMEM((B, H), jnp.float32),
                    pltpu.VMEM((B, 1), jnp.float32)],
    compiler_params=pltpu.CompilerParams(
        dimension_semantics=("arbitrary",)),
)


def kernel(x, edge_index, node_graph_ids, a,
           Wself0, Wneigh0, b0, Wself1, Wneigh1, b1, Wself2, Wneigh2, b2,
           Wc1, bc1, Wc2, bc2, Wc3, bc3):
    pad = EPAD - E
    src = jnp.concatenate([edge_index[0], jnp.zeros((pad,), jnp.int32)])
    dst = jnp.concatenate(
        [edge_index[1], (jnp.arange(pad, dtype=jnp.int32) % 16) + N])
    src = src.reshape(NW, NCHUNK, CH)
    dst = dst.reshape(NW, NCHUNK, CH)
    zeros32 = jnp.zeros((NP, H), jnp.float32)
    zeros8 = jnp.zeros((NP, 8), jnp.float32)
    ones8 = jnp.ones((CH, 8), jnp.float32)

    g0, xs = _tc1(x, Wneigh0, Wself0)
    s0p, degp = _sc_scatter_deg(src, dst, g0, zeros32, zeros8, ones8)
    h1s, g1, dinv = _tc2(xs, s0p[:, :N], degp[:, :N], b0.reshape(1, H),
                         Wself1, Wneigh1)
    s1p = _sc_scatter(src, dst, g1, zeros32)
    h2s, g2 = _tc3(h1s, s1p[:, :N], dinv, b1.reshape(1, H), Wself2, Wneigh2)
    s2p = _sc_scatter(src, dst, g2, zeros32)

    mask = (node_graph_ids[:, None]
            == jnp.arange(B, dtype=jnp.int32)[None, :]).astype(jnp.float32)
    a2 = jnp.squeeze(a, -1)
    q = _tc4(h2s, s2p[:, :N], dinv, b2.reshape(1, H), mask, a2,
             Wc1[:H], Wc1[H:], bc1.reshape(1, NET), Wc2,
             bc2.reshape(1, NET), Wc3, bc3.reshape(1, R_DIM))
    return q


# speedup vs baseline: 20.8857x; 1.1687x over previous
import functools

import jax
import jax.numpy as jnp
from jax import lax
from jax.experimental import pallas as pl
from jax.experimental.pallas import tpu as pltpu
from jax.experimental.pallas import tpu_sc as plsc

N = 10000
E = 320000
D_IN = 128
H = 32
B = 16
A_DIM = 8
R_DIM = 4
NET = 256

NC = 2
NS = 16
NW = NC * NS
CH = 128
NCHUNK = 80
EPT = NCHUNK * CH
EPAD = NW * EPT
NP = 10112
RZ = NP // NS

NBUF = 4
_DO_SCATTER = True
_DO_GATHER = True
_GATHER_SPMEM = True

RB = 2000
NG = N // RB

_MESH = plsc.VectorSubcoreMesh(
    core_axis_name="c", subcore_axis_name="s", num_cores=NC, num_subcores=NS)


def _sc_edge_body(with_deg, src_hbm, dst_hbm, g_hbm, zeros_hbm, *rest):
    if with_deg:
        (zeros8_hbm, ones8_hbm, out_hbm, deg_hbm,
         acc_sp, deg_sp, g_sp, src_v, dst_v, rows_v, ones_v, sems, dsem) = rest
    else:
        (out_hbm, acc_sp, g_sp, src_v, dst_v, rows_v, sems) = rest
        dsem = None
    c = lax.axis_index("c")
    s = lax.axis_index("s")
    wid = c * NS + s

    pltpu.sync_copy(zeros_hbm.at[pl.ds(s * RZ, RZ)], acc_sp.at[pl.ds(s * RZ, RZ)])
    pltpu.sync_copy(src_hbm.at[wid], src_v)
    pltpu.sync_copy(dst_hbm.at[wid], dst_v)
    if _GATHER_SPMEM:
        @pl.when(s < NS - 1)
        def _stage():
            pltpu.sync_copy(g_hbm.at[pl.ds(s * RZ, RZ)],
                            g_sp.at[pl.ds(s * RZ, RZ)])

        @pl.when(s == NS - 1)
        def _stage_last():
            pltpu.sync_copy(g_hbm.at[pl.ds((NS - 1) * RZ, N - (NS - 1) * RZ)],
                            g_sp.at[pl.ds((NS - 1) * RZ, N - (NS - 1) * RZ)])
        g_src = g_sp
    else:
        g_src = g_hbm
    if with_deg:
        pltpu.sync_copy(zeros8_hbm.at[pl.ds(s * RZ, RZ)], deg_sp.at[pl.ds(s * RZ, RZ)])
        pltpu.sync_copy(ones8_hbm, ones_v)
    plsc.subcore_barrier()

    gsems, ssems = sems
    if _DO_GATHER:
        pltpu.async_copy(g_src.at[src_v.at[0]], rows_v.at[0], gsems.at[0])
        pltpu.async_copy(g_src.at[src_v.at[1]], rows_v.at[1], gsems.at[1])

    def outer(jj, carry):
        for b in range(NBUF):
            i = jj * NBUF + b
            bn = (b + 2) % NBUF

            if _DO_SCATTER:
                @pl.when(jnp.logical_and(i >= 2, i + 2 < NCHUNK))
                def _drain():
                    pltpu.make_async_copy(
                        rows_v.at[bn], acc_sp.at[dst_v.at[i]], ssems.at[bn]).wait()

            if _DO_GATHER:
                @pl.when(i + 2 < NCHUNK)
                def _fire():
                    pltpu.async_copy(
                        g_src.at[src_v.at[i + 2]], rows_v.at[bn], gsems.at[bn])

                pltpu.make_async_copy(
                    g_src.at[src_v.at[i]], rows_v.at[b], gsems.at[b]).wait()
            if _DO_SCATTER:
                pltpu.async_copy(rows_v.at[b], acc_sp.at[dst_v.at[i]],
                                 ssems.at[b], add=True)
            if with_deg and _DO_SCATTER:
                pltpu.async_copy(ones_v, deg_sp.at[dst_v.at[i]],
                                 dsem, add=True)
        return carry

    lax.fori_loop(0, NCHUNK // NBUF, outer, 0)

    for b in range(NBUF) if _DO_SCATTER else []:
        pltpu.make_async_copy(
            rows_v.at[b], acc_sp.at[dst_v.at[0]], ssems.at[b]).wait()
    if with_deg and _DO_SCATTER:
        def drain_deg(i, carry):
            pltpu.make_async_copy(
                ones_v, deg_sp.at[dst_v.at[0]], dsem).wait()
            return carry
        lax.fori_loop(0, NCHUNK, drain_deg, 0)
    plsc.subcore_barrier()

    pltpu.sync_copy(acc_sp.at[pl.ds(s * RZ, RZ)], out_hbm.at[c, pl.ds(s * RZ, RZ)])
    if with_deg:
        pltpu.sync_copy(deg_sp.at[pl.ds(s * RZ, RZ)], deg_hbm.at[c, pl.ds(s * RZ, RZ)])


_sc_scatter_deg = pl.kernel(
    functools.partial(_sc_edge_body, True),
    out_type=(
        jax.ShapeDtypeStruct((NC, NP, H), jnp.float32),
        jax.ShapeDtypeStruct((NC, NP, 8), jnp.float32),
    ),
    mesh=_MESH,
    compiler_params=pltpu.CompilerParams(use_tc_tiling_on_sc=False),
    scratch_types=[
        pltpu.VMEM_SHARED((NP, H), jnp.float32),
        pltpu.VMEM_SHARED((NP, 8), jnp.float32),
        pltpu.VMEM_SHARED((NP, H), jnp.float32),
        pltpu.VMEM((NCHUNK, CH), jnp.int32),
        pltpu.VMEM((NCHUNK, CH), jnp.int32),
        pltpu.VMEM((NBUF, CH, H), jnp.float32),
        pltpu.VMEM((CH, 8), jnp.float32),
        (pltpu.SemaphoreType.DMA((NBUF,)), pltpu.SemaphoreType.DMA((NBUF,))),
        pltpu.SemaphoreType.DMA,
    ],
)

_sc_scatter = pl.kernel(
    functools.partial(_sc_edge_body, False),
    out_type=jax.ShapeDtypeStruct((NC, NP, H), jnp.float32),
    mesh=_MESH,
    compiler_params=pltpu.CompilerParams(use_tc_tiling_on_sc=False),
    scratch_types=[
        pltpu.VMEM_SHARED((NP, H), jnp.float32),
        pltpu.VMEM_SHARED((NP, H), jnp.float32),
        pltpu.VMEM((NCHUNK, CH), jnp.int32),
        pltpu.VMEM((NCHUNK, CH), jnp.int32),
        pltpu.VMEM((NBUF, CH, H), jnp.float32),
        (pltpu.SemaphoreType.DMA((NBUF,)), pltpu.SemaphoreType.DMA((NBUF,))),
    ],
)


def _tc1_body(x_ref, wn_ref, ws_ref, g_ref, xs_ref):
    x = x_ref[...]
    g_ref[...] = jnp.dot(x, wn_ref[...], preferred_element_type=jnp.float32)
    xs_ref[...] = jnp.dot(x, ws_ref[...], preferred_element_type=jnp.float32)


def _tc2_body(xs_ref, sp_ref, degp_ref, b_ref, wsn_ref, wnn_ref,
              hs_ref, g_ref, dinv_ref):
    ssum = sp_ref[0] + sp_ref[1]
    deg = degp_ref[0, :, 0:1] + degp_ref[1, :, 0:1]
    dinv = 1.0 / jnp.maximum(deg, 1.0)
    h = jnp.maximum(xs_ref[...] + ssum * dinv + b_ref[...], 0.0)
    hs_ref[...] = jnp.dot(h, wsn_ref[...], preferred_element_type=jnp.float32)
    g_ref[...] = jnp.dot(h, wnn_ref[...], preferred_element_type=jnp.float32)
    dinv_ref[...] = dinv


def _tc3_body(hs_ref, sp_ref, dinv_ref, b_ref, wsn_ref, wnn_ref,
              hs2_ref, g2_ref):
    ssum = sp_ref[0] + sp_ref[1]
    h = jnp.maximum(hs_ref[...] + ssum * dinv_ref[...] + b_ref[...], 0.0)
    hs2_ref[...] = jnp.dot(h, wsn_ref[...], preferred_element_type=jnp.float32)
    g2_ref[...] = jnp.dot(h, wnn_ref[...], preferred_element_type=jnp.float32)


def _tc4_body(hs_ref, sp_ref, dinv_ref, b_ref, mask_ref, a2_ref,
              wc1h_ref, wc1a_ref, bc1_ref, wc2_ref, bc2_ref, wc3_ref, bc3_ref,
              q_ref, acc_ref, cnt_ref):
    i = pl.program_id(0)
    ssum = sp_ref[0] + sp_ref[1]
    h3 = jnp.maximum(hs_ref[...] + ssum * dinv_ref[...] + b_ref[...], 0.0)
    m = mask_ref[...]
    dn = (((0,), (0,)), ((), ()))
    pacc = lax.dot_general(m, h3, dn,
                           preferred_element_type=jnp.float32)
    pcnt = lax.dot_general(m, jnp.ones((RB, 1), jnp.float32), dn,
                           preferred_element_type=jnp.float32)

    @pl.when(i == 0)
    def _init():
        acc_ref[...] = pacc
        cnt_ref[...] = pcnt

    @pl.when(i > 0)
    def _accum():
        acc_ref[...] += pacc
        cnt_ref[...] += pcnt

    @pl.when(i == NG - 1)
    def _finish():
        nf = acc_ref[...] / jnp.maximum(cnt_ref[...], 1.0)
        z = jnp.dot(nf, wc1h_ref[...], preferred_element_type=jnp.float32)
        z = z + jnp.dot(a2_ref[...], wc1a_ref[...],
                        preferred_element_type=jnp.float32) + bc1_ref[...]
        z = jnp.maximum(z, 0.0)
        z = jnp.maximum(jnp.dot(z, wc2_ref[...],
                                preferred_element_type=jnp.float32)
                        + bc2_ref[...], 0.0)
        q_ref[...] = jnp.dot(z, wc3_ref[...],
                             preferred_element_type=jnp.float32) + bc3_ref[...]


def _row_spec(cols):
    return pl.BlockSpec((RB, cols), lambda i: (i, 0))


def _full_spec(shape):
    nd = len(shape)
    return pl.BlockSpec(shape, lambda i, _nd=nd: (0,) * _nd)


def _part_spec(cols):
    return pl.BlockSpec((NC, RB, cols), lambda i: (0, i, 0))


_tc1 = pl.pallas_call(
    _tc1_body,
    grid=(NG,),
    in_specs=[_row_spec(D_IN), _full_spec((D_IN, H)), _full_spec((D_IN, H))],
    out_specs=(_row_spec(H), _row_spec(H)),
    out_shape=(jax.ShapeDtypeStruct((N, H), jnp.float32),
               jax.ShapeDtypeStruct((N, H), jnp.float32)),
    compiler_params=pltpu.CompilerParams(
        dimension_semantics=("parallel",)),
)

_tc2 = pl.pallas_call(
    _tc2_body,
    grid=(NG,),
    in_specs=[_row_spec(H), _part_spec(H), _part_spec(8), _full_spec((1, H)),
              _full_spec((H, H)), _full_spec((H, H))],
    out_specs=(_row_spec(H), _row_spec(H), _row_spec(1)),
    out_shape=(jax.ShapeDtypeStruct((N, H), jnp.float32),
               jax.ShapeDtypeStruct((N, H), jnp.float32),
               jax.ShapeDtypeStruct((N, 1), jnp.float32)),
    compiler_params=pltpu.CompilerParams(
        dimension_semantics=("parallel",)),
)

_tc3 = pl.pallas_call(
    _tc3_body,
    grid=(NG,),
    in_specs=[_row_spec(H), _part_spec(H), _row_spec(1), _full_spec((1, H)),
              _full_spec((H, H)), _full_spec((H, H))],
    out_specs=(_row_spec(H), _row_spec(H)),
    out_shape=(jax.ShapeDtypeStruct((N, H), jnp.float32),
               jax.ShapeDtypeStruct((N, H), jnp.float32)),
    compiler_params=pltpu.CompilerParams(
        dimension_semantics=("parallel",)),
)

_tc4 = pl.pallas_call(
    _tc4_body,
    grid=(NG,),
    in_specs=[_row_spec(H), _part_spec(H), _row_spec(1), _full_spec((1, H)),
              _row_spec(B), _full_spec((B, A_DIM)),
              _full_spec((H, NET)), _full_spec((A_DIM, NET)),
              _full_spec((1, NET)), _full_spec((NET, NET)),
              _full_spec((1, NET)), _full_spec((NET, R_DIM)),
              _full_spec((1, R_DIM))],
    out_specs=_full_spec((B, R_DIM)),
    out_shape=jax.ShapeDtypeStruct((B, R_DIM), jnp.float32),
    scratch_shapes=[pltpu.VMEM((B, H), jnp.float32),
                    pltpu.VMEM((B, 1), jnp.float32)],
    compiler_params=pltpu.CompilerParams(
        dimension_semantics=("arbitrary",)),
)


def kernel(x, edge_index, node_graph_ids, a,
           Wself0, Wneigh0, b0, Wself1, Wneigh1, b1, Wself2, Wneigh2, b2,
           Wc1, bc1, Wc2, bc2, Wc3, bc3):
    pad = EPAD - E
    src = jnp.concatenate([edge_index[0], jnp.zeros((pad,), jnp.int32)])
    dst = jnp.concatenate(
        [edge_index[1], (jnp.arange(pad, dtype=jnp.int32) % 16) + N])
    src = src.reshape(NW, NCHUNK, CH)
    dst = dst.reshape(NW, NCHUNK, CH)
    zeros32 = jnp.zeros((NP, H), jnp.float32)
    zeros8 = jnp.zeros((NP, 8), jnp.float32)
    ones8 = jnp.ones((CH, 8), jnp.float32)

    g0, xs = _tc1(x, Wneigh0, Wself0)
    s0p, degp = _sc_scatter_deg(src, dst, g0, zeros32, zeros8, ones8)
    h1s, g1, dinv = _tc2(xs, s0p, degp, b0.reshape(1, H),
                         Wself1, Wneigh1)
    s1p = _sc_scatter(src, dst, g1, zeros32)
    h2s, g2 = _tc3(h1s, s1p, dinv, b1.reshape(1, H), Wself2, Wneigh2)
    s2p = _sc_scatter(src, dst, g2, zeros32)

    mask = (node_graph_ids[:, None]
            == jnp.arange(B, dtype=jnp.int32)[None, :]).astype(jnp.float32)
    a2 = jnp.squeeze(a, -1)
    q = _tc4(h2s, s2p, dinv, b2.reshape(1, H), mask, a2,
             Wc1[:H], Wc1[H:], bc1.reshape(1, NET), Wc2,
             bc2.reshape(1, NET), Wc3, bc3.reshape(1, R_DIM))
    return q


# speedup vs baseline: 22.5502x; 1.0797x over previous
import functools

import jax
import jax.numpy as jnp
from jax import lax
from jax.experimental import pallas as pl
from jax.experimental.pallas import tpu as pltpu
from jax.experimental.pallas import tpu_sc as plsc

N = 10000
E = 320000
D_IN = 128
H = 32
B = 16
A_DIM = 8
R_DIM = 4
NET = 256
W128 = 128

NC = 2
NS = 16
NW = NC * NS
CH = 128
NCHUNK = 80
EPT = NCHUNK * CH
EPAD = NW * EPT
NP = 10112
RZ = NP // NS
NLAST = N - (NS - 1) * RZ

NBUF = 4

RB = 2000
NG = N // RB

_MESH = plsc.VectorSubcoreMesh(
    core_axis_name="c", subcore_axis_name="s", num_cores=NC, num_subcores=NS)


def _sc_edge_body(with_deg, src_hbm, dst_hbm, g_hbm, zeros_hbm, *rest):
    if with_deg:
        (ones_hbm, out_hbm,
         acc_sp, deg_sp, g_sp, src_v, dst_v, rows_v, ones_v, sems, dsem) = rest
    else:
        (out_hbm, acc_sp, g_sp, src_v, dst_v, rows_v, sems) = rest
    c = lax.axis_index("c")
    s = lax.axis_index("s")
    wid = c * NS + s

    pltpu.sync_copy(zeros_hbm.at[pl.ds(s * RZ, RZ)], acc_sp.at[pl.ds(s * RZ, RZ)])
    pltpu.sync_copy(src_hbm.at[wid], src_v)
    pltpu.sync_copy(dst_hbm.at[wid], dst_v)

    @pl.when(s < NS - 1)
    def _stage():
        pltpu.sync_copy(g_hbm.at[pl.ds(s * RZ, RZ), pl.ds(0, H)],
                        g_sp.at[pl.ds(s * RZ, RZ)])

    @pl.when(s == NS - 1)
    def _stage_last():
        pltpu.sync_copy(g_hbm.at[pl.ds((NS - 1) * RZ, NLAST), pl.ds(0, H)],
                        g_sp.at[pl.ds((NS - 1) * RZ, NLAST)])

    if with_deg:
        pltpu.sync_copy(zeros_hbm.at[pl.ds(s * RZ, RZ)],
                        deg_sp.at[pl.ds(s * RZ, RZ)])
        pltpu.sync_copy(ones_hbm, ones_v)
    plsc.subcore_barrier()

    gsems, ssems = sems
    pltpu.async_copy(g_sp.at[src_v.at[0]], rows_v.at[0], gsems.at[0])
    pltpu.async_copy(g_sp.at[src_v.at[1]], rows_v.at[1], gsems.at[1])

    def outer(jj, carry):
        for b in range(NBUF):
            i = jj * NBUF + b
            bn = (b + 2) % NBUF

            @pl.when(jnp.logical_and(i >= 2, i + 2 < NCHUNK))
            def _drain():
                pltpu.make_async_copy(
                    rows_v.at[bn], acc_sp.at[dst_v.at[i]], ssems.at[bn]).wait()

            @pl.when(i + 2 < NCHUNK)
            def _fire():
                pltpu.async_copy(
                    g_sp.at[src_v.at[i + 2]], rows_v.at[bn], gsems.at[bn])

            pltpu.make_async_copy(
                g_sp.at[src_v.at[i]], rows_v.at[b], gsems.at[b]).wait()
            pltpu.async_copy(rows_v.at[b], acc_sp.at[dst_v.at[i]],
                             ssems.at[b], add=True)
            if with_deg:
                pltpu.async_copy(ones_v, deg_sp.at[dst_v.at[i]],
                                 dsem, add=True)
        return carry

    lax.fori_loop(0, NCHUNK // NBUF, outer, 0)

    for b in range(NBUF):
        pltpu.make_async_copy(
            rows_v.at[b], acc_sp.at[dst_v.at[0]], ssems.at[b]).wait()
    if with_deg:
        def drain_deg(i, carry):
            pltpu.make_async_copy(
                ones_v, deg_sp.at[dst_v.at[0]], dsem).wait()
            return carry
        lax.fori_loop(0, NCHUNK, drain_deg, 0)
    plsc.subcore_barrier()

    pltpu.sync_copy(acc_sp.at[pl.ds(s * RZ, RZ)],
                    out_hbm.at[c, pl.ds(s * RZ, RZ), pl.ds(0, H)])
    if with_deg:
        pltpu.sync_copy(deg_sp.at[pl.ds(s * RZ, RZ)],
                        out_hbm.at[c, pl.ds(s * RZ, RZ), pl.ds(H, H)])


_sc_scatter_deg = pl.kernel(
    functools.partial(_sc_edge_body, True),
    out_type=jax.ShapeDtypeStruct((NC, NP, W128), jnp.float32),
    mesh=_MESH,
    compiler_params=pltpu.CompilerParams(use_tc_tiling_on_sc=False),
    scratch_types=[
        pltpu.VMEM_SHARED((NP, H), jnp.float32),
        pltpu.VMEM_SHARED((NP, H), jnp.float32),
        pltpu.VMEM_SHARED((N, H), jnp.float32),
        pltpu.VMEM((NCHUNK, CH), jnp.int32),
        pltpu.VMEM((NCHUNK, CH), jnp.int32),
        pltpu.VMEM((NBUF, CH, H), jnp.float32),
        pltpu.VMEM((CH, H), jnp.float32),
        (pltpu.SemaphoreType.DMA((NBUF,)), pltpu.SemaphoreType.DMA((NBUF,))),
        pltpu.SemaphoreType.DMA,
    ],
)

_sc_scatter = pl.kernel(
    functools.partial(_sc_edge_body, False),
    out_type=jax.ShapeDtypeStruct((NC, NP, W128), jnp.float32),
    mesh=_MESH,
    compiler_params=pltpu.CompilerParams(use_tc_tiling_on_sc=False),
    scratch_types=[
        pltpu.VMEM_SHARED((NP, H), jnp.float32),
        pltpu.VMEM_SHARED((N, H), jnp.float32),
        pltpu.VMEM((NCHUNK, CH), jnp.int32),
        pltpu.VMEM((NCHUNK, CH), jnp.int32),
        pltpu.VMEM((NBUF, CH, H), jnp.float32),
        (pltpu.SemaphoreType.DMA((NBUF,)), pltpu.SemaphoreType.DMA((NBUF,))),
    ],
)


def _col32():
    return (lax.broadcasted_iota(jnp.int32, (1, W128), 1) == H).astype(
        jnp.float32)


def _tc1_body(x_ref, wn_ref, ws_ref, g_ref, xs_ref):
    x = x_ref[...]
    g_ref[...] = jnp.dot(x, wn_ref[...], preferred_element_type=jnp.float32)
    xs_ref[...] = jnp.dot(x, ws_ref[...], preferred_element_type=jnp.float32)


def _tc2_body(xs_ref, sd_ref, b_ref, wsn_ref, wnn_ref, hs_ref, g_ref):
    ssum = sd_ref[0, :, :H] + sd_ref[1, :, :H]
    deg = sd_ref[0, :, H:H + 1] + sd_ref[1, :, H:H + 1]
    dinv = 1.0 / jnp.maximum(deg, 1.0)
    h = jnp.maximum(xs_ref[:, :H] + ssum * dinv + b_ref[...], 0.0)
    hs_ref[...] = (jnp.dot(h, wsn_ref[...], preferred_element_type=jnp.float32)
                   + dinv * _col32())
    g_ref[...] = jnp.dot(h, wnn_ref[...], preferred_element_type=jnp.float32)


def _tc3_body(hs_ref, sp_ref, b_ref, wsn_ref, wnn_ref, hs2_ref, g2_ref):
    ssum = sp_ref[0, :, :H] + sp_ref[1, :, :H]
    dinv = hs_ref[:, H:H + 1]
    h = jnp.maximum(hs_ref[:, :H] + ssum * dinv + b_ref[...], 0.0)
    hs2_ref[...] = (jnp.dot(h, wsn_ref[...], preferred_element_type=jnp.float32)
                    + dinv * _col32())
    g2_ref[...] = jnp.dot(h, wnn_ref[...], preferred_element_type=jnp.float32)


def _tc4_body(hs_ref, sp_ref, b_ref, mask_ref, a2_ref,
              wc1h_ref, wc1a_ref, bc1_ref, wc2_ref, bc2_ref, wc3_ref, bc3_ref,
              q_ref, acc_ref, cnt_ref):
    i = pl.program_id(0)
    ssum = sp_ref[0, :, :H] + sp_ref[1, :, :H]
    dinv = hs_ref[:, H:H + 1]
    h3 = jnp.maximum(hs_ref[:, :H] + ssum * dinv + b_ref[...], 0.0)
    m = mask_ref[:, :B]
    dn = (((0,), (0,)), ((), ()))
    pacc = lax.dot_general(m, h3, dn,
                           preferred_element_type=jnp.float32)
    pcnt = lax.dot_general(m, jnp.ones((RB, 1), jnp.float32), dn,
                           preferred_element_type=jnp.float32)

    @pl.when(i == 0)
    def _init():
        acc_ref[...] = pacc
        cnt_ref[...] = pcnt

    @pl.when(i > 0)
    def _accum():
        acc_ref[...] += pacc
        cnt_ref[...] += pcnt

    @pl.when(i == NG - 1)
    def _finish():
        nf = acc_ref[...] / jnp.maximum(cnt_ref[...], 1.0)
        z = jnp.dot(nf, wc1h_ref[...], preferred_element_type=jnp.float32)
        z = z + jnp.dot(a2_ref[...], wc1a_ref[...],
                        preferred_element_type=jnp.float32) + bc1_ref[...]
        z = jnp.maximum(z, 0.0)
        z = jnp.maximum(jnp.dot(z, wc2_ref[...],
                                preferred_element_type=jnp.float32)
                        + bc2_ref[...], 0.0)
        q_ref[...] = jnp.dot(z, wc3_ref[...],
                             preferred_element_type=jnp.float32) + bc3_ref[...]


def _row_spec(cols):
    return pl.BlockSpec((RB, cols), lambda i: (i, 0))


def _full_spec(shape):
    nd = len(shape)
    return pl.BlockSpec(shape, lambda i, _nd=nd: (0,) * _nd)


def _part_spec():
    return pl.BlockSpec((NC, RB, W128), lambda i: (0, i, 0))


_tc1 = pl.pallas_call(
    _tc1_body,
    grid=(NG,),
    in_specs=[_row_spec(D_IN), _full_spec((D_IN, W128)),
              _full_spec((D_IN, W128))],
    out_specs=(_row_spec(W128), _row_spec(W128)),
    out_shape=(jax.ShapeDtypeStruct((N, W128), jnp.float32),
               jax.ShapeDtypeStruct((N, W128), jnp.float32)),
    compiler_params=pltpu.CompilerParams(
        dimension_semantics=("parallel",)),
)

_tc2 = pl.pallas_call(
    _tc2_body,
    grid=(NG,),
    in_specs=[_row_spec(W128), _part_spec(), _full_spec((1, H)),
              _full_spec((H, W128)), _full_spec((H, W128))],
    out_specs=(_row_spec(W128), _row_spec(W128)),
    out_shape=(jax.ShapeDtypeStruct((N, W128), jnp.float32),
               jax.ShapeDtypeStruct((N, W128), jnp.float32)),
    compiler_params=pltpu.CompilerParams(
        dimension_semantics=("parallel",)),
)

_tc3 = pl.pallas_call(
    _tc3_body,
    grid=(NG,),
    in_specs=[_row_spec(W128), _part_spec(), _full_spec((1, H)),
              _full_spec((H, W128)), _full_spec((H, W128))],
    out_specs=(_row_spec(W128), _row_spec(W128)),
    out_shape=(jax.ShapeDtypeStruct((N, W128), jnp.float32),
               jax.ShapeDtypeStruct((N, W128), jnp.float32)),
    compiler_params=pltpu.CompilerParams(
        dimension_semantics=("parallel",)),
)

_tc4 = pl.pallas_call(
    _tc4_body,
    grid=(NG,),
    in_specs=[_row_spec(W128), _part_spec(), _full_spec((1, H)),
              _row_spec(W128), _full_spec((B, A_DIM)),
              _full_spec((H, NET)), _full_spec((A_DIM, NET)),
              _full_spec((1, NET)), _full_spec((NET, NET)),
              _full_spec((1, NET)), _full_spec((NET, R_DIM)),
              _full_spec((1, R_DIM))],
    out_specs=_full_spec((B, R_DIM)),
    out_shape=jax.ShapeDtypeStruct((B, R_DIM), jnp.float32),
    scratch_shapes=[pltpu.VMEM((B, H), jnp.float32),
                    pltpu.VMEM((B, 1), jnp.float32)],
    compiler_params=pltpu.CompilerParams(
        dimension_semantics=("arbitrary",)),
)


def _pad128(w):
    return jnp.pad(w, ((0, 0), (0, W128 - w.shape[1])))


def kernel(x, edge_index, node_graph_ids, a,
           Wself0, Wneigh0, b0, Wself1, Wneigh1, b1, Wself2, Wneigh2, b2,
           Wc1, bc1, Wc2, bc2, Wc3, bc3):
    pad = EPAD - E
    src = jnp.concatenate([edge_index[0], jnp.zeros((pad,), jnp.int32)])
    dst = jnp.concatenate(
        [edge_index[1], (jnp.arange(pad, dtype=jnp.int32) % 16) + N])
    src = src.reshape(NW, NCHUNK, CH)
    dst = dst.reshape(NW, NCHUNK, CH)
    zeros = jnp.zeros((NP, H), jnp.float32)
    ones = jnp.ones((CH, H), jnp.float32)

    g0, xs = _tc1(x, _pad128(Wneigh0), _pad128(Wself0))
    sd0 = _sc_scatter_deg(src, dst, g0, zeros, ones)
    h1s, g1 = _tc2(xs, sd0, b0.reshape(1, H),
                   _pad128(Wself1), _pad128(Wneigh1))
    s1 = _sc_scatter(src, dst, g1, zeros)
    h2s, g2 = _tc3(h1s, s1, b1.reshape(1, H),
                   _pad128(Wself2), _pad128(Wneigh2))
    s2 = _sc_scatter(src, dst, g2, zeros)

    mask = (node_graph_ids[:, None]
            == jnp.arange(W128, dtype=jnp.int32)[None, :]).astype(jnp.float32)
    a2 = jnp.squeeze(a, -1)
    q = _tc4(h2s, s2, b2.reshape(1, H), mask, a2,
             Wc1[:H], Wc1[H:], bc1.reshape(1, NET), Wc2,
             bc2.reshape(1, NET), Wc3, bc3.reshape(1, R_DIM))
    return q


# speedup vs baseline: 24.1426x; 1.0706x over previous
import functools

import jax
import jax.numpy as jnp
from jax import lax
from jax.experimental import pallas as pl
from jax.experimental.pallas import tpu as pltpu
from jax.experimental.pallas import tpu_sc as plsc

N = 10000
E = 320000
D_IN = 128
H = 32
B = 16
A_DIM = 8
R_DIM = 4
NET = 256
W128 = 128
DW = 8

NC = 2
NS = 16
NW = NC * NS
CH = 128
NCHUNK = 80
EPT = NCHUNK * CH
EPAD = NW * EPT
NP = 10112
RZ = NP // NS
NLAST = N - (NS - 1) * RZ

NBUF = 4

RB = 2000
NG = N // RB

_MESH = plsc.VectorSubcoreMesh(
    core_axis_name="c", subcore_axis_name="s", num_cores=NC, num_subcores=NS)


def _sc_edge_body(with_deg, src_hbm, dst_hbm, g_hbm, zeros_hbm, *rest):
    if with_deg:
        (ones_hbm, out_hbm,
         acc_sp, deg_sp, g_sp, src_v, dst_v, rows_v, ones_v, sems, dsem) = rest
    else:
        (out_hbm, acc_sp, g_sp, src_v, dst_v, rows_v, sems) = rest
    c = lax.axis_index("c")
    s = lax.axis_index("s")
    wid = c * NS + s

    pltpu.sync_copy(zeros_hbm.at[pl.ds(s * RZ, RZ)], acc_sp.at[pl.ds(s * RZ, RZ)])
    pltpu.sync_copy(src_hbm.at[wid], src_v)
    pltpu.sync_copy(dst_hbm.at[wid], dst_v)

    @pl.when(s < NS - 1)
    def _stage():
        pltpu.sync_copy(g_hbm.at[pl.ds(s * RZ, RZ), pl.ds(0, H)],
                        g_sp.at[pl.ds(s * RZ, RZ)])

    @pl.when(s == NS - 1)
    def _stage_last():
        pltpu.sync_copy(g_hbm.at[pl.ds((NS - 1) * RZ, NLAST), pl.ds(0, H)],
                        g_sp.at[pl.ds((NS - 1) * RZ, NLAST)])

    if with_deg:
        pltpu.sync_copy(zeros_hbm.at[pl.ds(s * RZ, RZ), pl.ds(0, DW)],
                        deg_sp.at[pl.ds(s * RZ, RZ)])
        pltpu.sync_copy(ones_hbm, ones_v)
    plsc.subcore_barrier()

    gsems, ssems = sems
    pltpu.async_copy(g_sp.at[src_v.at[0]], rows_v.at[0], gsems.at[0])
    pltpu.async_copy(g_sp.at[src_v.at[1]], rows_v.at[1], gsems.at[1])

    def outer(jj, carry):
        for b in range(NBUF):
            i = jj * NBUF + b
            bn = (b + 2) % NBUF

            @pl.when(jnp.logical_and(i >= 2, i + 2 < NCHUNK))
            def _drain():
                pltpu.make_async_copy(
                    rows_v.at[bn], acc_sp.at[dst_v.at[i]], ssems.at[bn]).wait()

            @pl.when(i + 2 < NCHUNK)
            def _fire():
                pltpu.async_copy(
                    g_sp.at[src_v.at[i + 2]], rows_v.at[bn], gsems.at[bn])

            pltpu.make_async_copy(
                g_sp.at[src_v.at[i]], rows_v.at[b], gsems.at[b]).wait()
            pltpu.async_copy(rows_v.at[b], acc_sp.at[dst_v.at[i]],
                             ssems.at[b], add=True)
            if with_deg:
                pltpu.async_copy(ones_v, deg_sp.at[dst_v.at[i]],
                                 dsem, add=True)
        return carry

    lax.fori_loop(0, NCHUNK // NBUF, outer, 0)

    for b in range(NBUF):
        pltpu.make_async_copy(
            rows_v.at[b], acc_sp.at[dst_v.at[0]], ssems.at[b]).wait()
    if with_deg:
        def drain_deg(i, carry):
            pltpu.make_async_copy(
                ones_v, deg_sp.at[dst_v.at[0]], dsem).wait()
            return carry
        lax.fori_loop(0, NCHUNK, drain_deg, 0)
    plsc.subcore_barrier()

    pltpu.sync_copy(acc_sp.at[pl.ds(s * RZ, RZ)],
                    out_hbm.at[c, pl.ds(s * RZ, RZ), pl.ds(0, H)])
    if with_deg:
        pltpu.sync_copy(deg_sp.at[pl.ds(s * RZ, RZ)],
                        out_hbm.at[c, pl.ds(s * RZ, RZ), pl.ds(H, DW)])


_sc_scatter_deg = pl.kernel(
    functools.partial(_sc_edge_body, True),
    out_type=jax.ShapeDtypeStruct((NC, NP, W128), jnp.float32),
    mesh=_MESH,
    compiler_params=pltpu.CompilerParams(use_tc_tiling_on_sc=False),
    scratch_types=[
        pltpu.VMEM_SHARED((NP, H), jnp.float32),
        pltpu.VMEM_SHARED((NP, DW), jnp.float32),
        pltpu.VMEM_SHARED((N, H), jnp.float32),
        pltpu.VMEM((NCHUNK, CH), jnp.int32),
        pltpu.VMEM((NCHUNK, CH), jnp.int32),
        pltpu.VMEM((NBUF, CH, H), jnp.float32),
        pltpu.VMEM((CH, DW), jnp.float32),
        (pltpu.SemaphoreType.DMA((NBUF,)), pltpu.SemaphoreType.DMA((NBUF,))),
        pltpu.SemaphoreType.DMA,
    ],
)

_sc_scatter = pl.kernel(
    functools.partial(_sc_edge_body, False),
    out_type=jax.ShapeDtypeStruct((NC, NP, W128), jnp.float32),
    mesh=_MESH,
    compiler_params=pltpu.CompilerParams(use_tc_tiling_on_sc=False),
    scratch_types=[
        pltpu.VMEM_SHARED((NP, H), jnp.float32),
        pltpu.VMEM_SHARED((N, H), jnp.float32),
        pltpu.VMEM((NCHUNK, CH), jnp.int32),
        pltpu.VMEM((NCHUNK, CH), jnp.int32),
        pltpu.VMEM((NBUF, CH, H), jnp.float32),
        (pltpu.SemaphoreType.DMA((NBUF,)), pltpu.SemaphoreType.DMA((NBUF,))),
    ],
)


def _col32():
    return (lax.broadcasted_iota(jnp.int32, (1, W128), 1) == H).astype(
        jnp.float32)


def _tc1_body(x_ref, wn_ref, ws_ref, g_ref, xs_ref):
    x = x_ref[...]
    g_ref[...] = jnp.dot(x, wn_ref[...], preferred_element_type=jnp.float32)
    xs_ref[...] = jnp.dot(x, ws_ref[...], preferred_element_type=jnp.float32)


def _tc2_body(xs_ref, sd_ref, b_ref, wsn_ref, wnn_ref, hs_ref, g_ref):
    ssum = sd_ref[0, :, :H] + sd_ref[1, :, :H]
    deg = sd_ref[0, :, H:H + 1] + sd_ref[1, :, H:H + 1]
    dinv = 1.0 / jnp.maximum(deg, 1.0)
    h = jnp.maximum(xs_ref[:, :H] + ssum * dinv + b_ref[...], 0.0)
    hs_ref[...] = (jnp.dot(h, wsn_ref[...], preferred_element_type=jnp.float32)
                   + dinv * _col32())
    g_ref[...] = jnp.dot(h, wnn_ref[...], preferred_element_type=jnp.float32)


def _tc3_body(hs_ref, sp_ref, b_ref, wsn_ref, wnn_ref, hs2_ref, g2_ref):
    ssum = sp_ref[0, :, :H] + sp_ref[1, :, :H]
    dinv = hs_ref[:, H:H + 1]
    h = jnp.maximum(hs_ref[:, :H] + ssum * dinv + b_ref[...], 0.0)
    hs2_ref[...] = (jnp.dot(h, wsn_ref[...], preferred_element_type=jnp.float32)
                    + dinv * _col32())
    g2_ref[...] = jnp.dot(h, wnn_ref[...], preferred_element_type=jnp.float32)


def _tc4_body(hs_ref, sp_ref, b_ref, mask_ref, a2_ref,
              wc1h_ref, wc1a_ref, bc1_ref, wc2_ref, bc2_ref, wc3_ref, bc3_ref,
              q_ref, acc_ref, cnt_ref):
    i = pl.program_id(0)
    ssum = sp_ref[0, :, :H] + sp_ref[1, :, :H]
    dinv = hs_ref[:, H:H + 1]
    h3 = jnp.maximum(hs_ref[:, :H] + ssum * dinv + b_ref[...], 0.0)
    m = mask_ref[:, :B]
    dn = (((0,), (0,)), ((), ()))
    pacc = lax.dot_general(m, h3, dn,
                           preferred_element_type=jnp.float32)
    pcnt = lax.dot_general(m, jnp.ones((RB, 1), jnp.float32), dn,
                           preferred_element_type=jnp.float32)

    @pl.when(i == 0)
    def _init():
        acc_ref[...] = pacc
        cnt_ref[...] = pcnt

    @pl.when(i > 0)
    def _accum():
        acc_ref[...] += pacc
        cnt_ref[...] += pcnt

    @pl.when(i == NG - 1)
    def _finish():
        nf = acc_ref[...] / jnp.maximum(cnt_ref[...], 1.0)
        z = jnp.dot(nf, wc1h_ref[...], preferred_element_type=jnp.float32)
        z = z + jnp.dot(a2_ref[...], wc1a_ref[...],
                        preferred_element_type=jnp.float32) + bc1_ref[...]
        z = jnp.maximum(z, 0.0)
        z = jnp.maximum(jnp.dot(z, wc2_ref[...],
                                preferred_element_type=jnp.float32)
                        + bc2_ref[...], 0.0)
        q_ref[...] = jnp.dot(z, wc3_ref[...],
                             preferred_element_type=jnp.float32) + bc3_ref[...]


def _row_spec(cols):
    return pl.BlockSpec((RB, cols), lambda i: (i, 0))


def _full_spec(shape):
    nd = len(shape)
    return pl.BlockSpec(shape, lambda i, _nd=nd: (0,) * _nd)


def _part_spec():
    return pl.BlockSpec((NC, RB, W128), lambda i: (0, i, 0))


_tc1 = pl.pallas_call(
    _tc1_body,
    grid=(NG,),
    in_specs=[_row_spec(D_IN), _full_spec((D_IN, W128)),
              _full_spec((D_IN, W128))],
    out_specs=(_row_spec(W128), _row_spec(W128)),
    out_shape=(jax.ShapeDtypeStruct((N, W128), jnp.float32),
               jax.ShapeDtypeStruct((N, W128), jnp.float32)),
    compiler_params=pltpu.CompilerParams(
        dimension_semantics=("parallel",)),
)

_tc2 = pl.pallas_call(
    _tc2_body,
    grid=(NG,),
    in_specs=[_row_spec(W128), _part_spec(), _full_spec((1, H)),
              _full_spec((H, W128)), _full_spec((H, W128))],
    out_specs=(_row_spec(W128), _row_spec(W128)),
    out_shape=(jax.ShapeDtypeStruct((N, W128), jnp.float32),
               jax.ShapeDtypeStruct((N, W128), jnp.float32)),
    compiler_params=pltpu.CompilerParams(
        dimension_semantics=("parallel",)),
)

_tc3 = pl.pallas_call(
    _tc3_body,
    grid=(NG,),
    in_specs=[_row_spec(W128), _part_spec(), _full_spec((1, H)),
              _full_spec((H, W128)), _full_spec((H, W128))],
    out_specs=(_row_spec(W128), _row_spec(W128)),
    out_shape=(jax.ShapeDtypeStruct((N, W128), jnp.float32),
               jax.ShapeDtypeStruct((N, W128), jnp.float32)),
    compiler_params=pltpu.CompilerParams(
        dimension_semantics=("parallel",)),
)

_tc4 = pl.pallas_call(
    _tc4_body,
    grid=(NG,),
    in_specs=[_row_spec(W128), _part_spec(), _full_spec((1, H)),
              _row_spec(W128), _full_spec((B, A_DIM)),
              _full_spec((H, NET)), _full_spec((A_DIM, NET)),
              _full_spec((1, NET)), _full_spec((NET, NET)),
              _full_spec((1, NET)), _full_spec((NET, R_DIM)),
              _full_spec((1, R_DIM))],
    out_specs=_full_spec((B, R_DIM)),
    out_shape=jax.ShapeDtypeStruct((B, R_DIM), jnp.float32),
    scratch_shapes=[pltpu.VMEM((B, H), jnp.float32),
                    pltpu.VMEM((B, 1), jnp.float32)],
    compiler_params=pltpu.CompilerParams(
        dimension_semantics=("arbitrary",)),
)


def _pad128(w):
    return jnp.pad(w, ((0, 0), (0, W128 - w.shape[1])))


def kernel(x, edge_index, node_graph_ids, a,
           Wself0, Wneigh0, b0, Wself1, Wneigh1, b1, Wself2, Wneigh2, b2,
           Wc1, bc1, Wc2, bc2, Wc3, bc3):
    pad = EPAD - E
    src = jnp.concatenate([edge_index[0], jnp.zeros((pad,), jnp.int32)])
    dst = jnp.concatenate(
        [edge_index[1], (jnp.arange(pad, dtype=jnp.int32) % 16) + N])
    src = src.reshape(NW, NCHUNK, CH)
    dst = dst.reshape(NW, NCHUNK, CH)
    zeros = jnp.zeros((NP, H), jnp.float32)
    ones = jnp.ones((CH, DW), jnp.float32)

    g0, xs = _tc1(x, _pad128(Wneigh0), _pad128(Wself0))
    sd0 = _sc_scatter_deg(src, dst, g0, zeros, ones)
    h1s, g1 = _tc2(xs, sd0, b0.reshape(1, H),
                   _pad128(Wself1), _pad128(Wneigh1))
    s1 = _sc_scatter(src, dst, g1, zeros)
    h2s, g2 = _tc3(h1s, s1, b1.reshape(1, H),
                   _pad128(Wself2), _pad128(Wneigh2))
    s2 = _sc_scatter(src, dst, g2, zeros)

    mask = (node_graph_ids[:, None]
            == jnp.arange(W128, dtype=jnp.int32)[None, :]).astype(jnp.float32)
    a2 = jnp.squeeze(a, -1)
    q = _tc4(h2s, s2, b2.reshape(1, H), mask, a2,
             Wc1[:H], Wc1[H:], bc1.reshape(1, NET), Wc2,
             bc2.reshape(1, NET), Wc3, bc3.reshape(1, R_DIM))
    return q


# speedup vs baseline: 24.4805x; 1.0140x over previous
import functools

import jax
import jax.numpy as jnp
from jax import lax
from jax.experimental import pallas as pl
from jax.experimental.pallas import tpu as pltpu
from jax.experimental.pallas import tpu_sc as plsc

N = 10000
E = 320000
D_IN = 128
H = 32
B = 16
A_DIM = 8
R_DIM = 4
NET = 256
W128 = 128
DW = 8

NC = 2
NS = 16
NW = NC * NS
CH = 128
NCHUNK = 80
EPT = NCHUNK * CH
EPAD = NW * EPT
NP = 10112
RZ = NP // NS
NLAST = N - (NS - 1) * RZ

NBUF = 4

RB = 2000
NG = N // RB

_MESH = plsc.VectorSubcoreMesh(
    core_axis_name="c", subcore_axis_name="s", num_cores=NC, num_subcores=NS)


def _sc_edge_body(with_deg, src_hbm, dst_hbm, g_hbm, zeros_hbm, *rest):
    if with_deg:
        (ones_hbm, out_hbm,
         acc_sp, deg_sp, g_sp, src_v, dst_v, rows_v, ones_v, sems, dsem) = rest
    else:
        (out_hbm, acc_sp, g_sp, src_v, dst_v, rows_v, sems) = rest
    c = lax.axis_index("c")
    s = lax.axis_index("s")
    wid = c * NS + s

    pltpu.sync_copy(zeros_hbm.at[pl.ds(s * RZ, RZ)], acc_sp.at[pl.ds(s * RZ, RZ)])
    pltpu.sync_copy(src_hbm.at[wid], src_v)
    pltpu.sync_copy(dst_hbm.at[wid], dst_v)

    @pl.when(s < NS - 1)
    def _stage():
        pltpu.sync_copy(g_hbm.at[pl.ds(s * RZ, RZ), pl.ds(0, H)],
                        g_sp.at[pl.ds(s * RZ, RZ)])

    @pl.when(s == NS - 1)
    def _stage_last():
        pltpu.sync_copy(g_hbm.at[pl.ds((NS - 1) * RZ, NLAST), pl.ds(0, H)],
                        g_sp.at[pl.ds((NS - 1) * RZ, NLAST)])

    if with_deg:
        pltpu.sync_copy(zeros_hbm.at[pl.ds(s * RZ, RZ), pl.ds(0, DW)],
                        deg_sp.at[pl.ds(s * RZ, RZ)])
        pltpu.sync_copy(ones_hbm, ones_v)
    plsc.subcore_barrier()

    gsems, ssems = sems
    pltpu.async_copy(g_sp.at[src_v.at[0]], rows_v.at[0], gsems.at[0])
    pltpu.async_copy(g_sp.at[src_v.at[1]], rows_v.at[1], gsems.at[1])

    def outer(jj, carry):
        for b in range(NBUF):
            i = jj * NBUF + b
            bn = (b + 2) % NBUF

            @pl.when(jnp.logical_and(i >= 2, i + 2 < NCHUNK))
            def _drain():
                pltpu.make_async_copy(
                    rows_v.at[bn], acc_sp.at[dst_v.at[i]], ssems.at[bn]).wait()

            @pl.when(i + 2 < NCHUNK)
            def _fire():
                pltpu.async_copy(
                    g_sp.at[src_v.at[i + 2]], rows_v.at[bn], gsems.at[bn])

            pltpu.make_async_copy(
                g_sp.at[src_v.at[i]], rows_v.at[b], gsems.at[b]).wait()
            pltpu.async_copy(rows_v.at[b], acc_sp.at[dst_v.at[i]],
                             ssems.at[b], add=True)
            if with_deg:
                pltpu.async_copy(ones_v, deg_sp.at[dst_v.at[i]],
                                 dsem, add=True)
        return carry

    lax.fori_loop(0, NCHUNK // NBUF, outer, 0)

    for b in range(NBUF):
        pltpu.make_async_copy(
            rows_v.at[b], acc_sp.at[dst_v.at[0]], ssems.at[b]).wait()
    if with_deg:
        def drain_deg(i, carry):
            pltpu.make_async_copy(
                ones_v, deg_sp.at[dst_v.at[0]], dsem).wait()
            return carry
        lax.fori_loop(0, NCHUNK, drain_deg, 0)
    plsc.subcore_barrier()

    for cc in range(NC):
        @pl.when(c == cc)
        def _copy_out(cc=cc):
            pltpu.sync_copy(acc_sp.at[pl.ds(s * RZ, RZ)],
                            out_hbm.at[pl.ds(s * RZ, RZ), pl.ds(cc * 64, H)])
            if with_deg:
                pltpu.sync_copy(
                    deg_sp.at[pl.ds(s * RZ, RZ)],
                    out_hbm.at[pl.ds(s * RZ, RZ), pl.ds(cc * 64 + H, DW)])


_sc_scatter_deg = pl.kernel(
    functools.partial(_sc_edge_body, True),
    out_type=jax.ShapeDtypeStruct((NP, W128), jnp.float32),
    mesh=_MESH,
    compiler_params=pltpu.CompilerParams(use_tc_tiling_on_sc=False),
    scratch_types=[
        pltpu.VMEM_SHARED((NP, H), jnp.float32),
        pltpu.VMEM_SHARED((NP, DW), jnp.float32),
        pltpu.VMEM_SHARED((N, H), jnp.float32),
        pltpu.VMEM((NCHUNK, CH), jnp.int32),
        pltpu.VMEM((NCHUNK, CH), jnp.int32),
        pltpu.VMEM((NBUF, CH, H), jnp.float32),
        pltpu.VMEM((CH, DW), jnp.float32),
        (pltpu.SemaphoreType.DMA((NBUF,)), pltpu.SemaphoreType.DMA((NBUF,))),
        pltpu.SemaphoreType.DMA,
    ],
)

_sc_scatter = pl.kernel(
    functools.partial(_sc_edge_body, False),
    out_type=jax.ShapeDtypeStruct((NP, W128), jnp.float32),
    mesh=_MESH,
    compiler_params=pltpu.CompilerParams(use_tc_tiling_on_sc=False),
    scratch_types=[
        pltpu.VMEM_SHARED((NP, H), jnp.float32),
        pltpu.VMEM_SHARED((N, H), jnp.float32),
        pltpu.VMEM((NCHUNK, CH), jnp.int32),
        pltpu.VMEM((NCHUNK, CH), jnp.int32),
        pltpu.VMEM((NBUF, CH, H), jnp.float32),
        (pltpu.SemaphoreType.DMA((NBUF,)), pltpu.SemaphoreType.DMA((NBUF,))),
    ],
)


def _col32():
    return (lax.broadcasted_iota(jnp.int32, (1, W128), 1) == H).astype(
        jnp.float32)


def _tc1_body(x_ref, wn_ref, g_ref):
    g_ref[...] = jnp.dot(x_ref[...], wn_ref[...],
                         preferred_element_type=jnp.float32)


def _tc2_body(x_ref, sd_ref, ws0_ref, b_ref, wsn_ref, wnn_ref, hs_ref, g_ref):
    ssum = sd_ref[:, :H] + sd_ref[:, 64:64 + H]
    deg = sd_ref[:, H:H + 1] + sd_ref[:, 96:97]
    dinv = 1.0 / jnp.maximum(deg, 1.0)
    xs = jnp.dot(x_ref[...], ws0_ref[...], preferred_element_type=jnp.float32)
    h = jnp.maximum(xs + ssum * dinv + b_ref[...], 0.0)
    hs_ref[...] = (jnp.dot(h, wsn_ref[...], preferred_element_type=jnp.float32)
                   + dinv * _col32())
    g_ref[...] = jnp.dot(h, wnn_ref[...], preferred_element_type=jnp.float32)


def _tc3_body(hs_ref, sp_ref, b_ref, wsn_ref, wnn_ref, hs2_ref, g2_ref):
    ssum = sp_ref[:, :H] + sp_ref[:, 64:64 + H]
    dinv = hs_ref[:, H:H + 1]
    h = jnp.maximum(hs_ref[:, :H] + ssum * dinv + b_ref[...], 0.0)
    hs2_ref[...] = (jnp.dot(h, wsn_ref[...], preferred_element_type=jnp.float32)
                    + dinv * _col32())
    g2_ref[...] = jnp.dot(h, wnn_ref[...], preferred_element_type=jnp.float32)


def _tc4_body(hs_ref, sp_ref, b_ref, mask_ref, a2_ref,
              wc1h_ref, wc1a_ref, bc1_ref, wc2_ref, bc2_ref, wc3_ref, bc3_ref,
              q_ref, acc_ref, cnt_ref):
    i = pl.program_id(0)
    ssum = sp_ref[:, :H] + sp_ref[:, 64:64 + H]
    dinv = hs_ref[:, H:H + 1]
    h3 = jnp.maximum(hs_ref[:, :H] + ssum * dinv + b_ref[...], 0.0)
    m = mask_ref[:, :B]
    dn = (((0,), (0,)), ((), ()))
    pacc = lax.dot_general(m, h3, dn,
                           preferred_element_type=jnp.float32)
    pcnt = lax.dot_general(m, jnp.ones((RB, 1), jnp.float32), dn,
                           preferred_element_type=jnp.float32)

    @pl.when(i == 0)
    def _init():
        acc_ref[...] = pacc
        cnt_ref[...] = pcnt

    @pl.when(i > 0)
    def _accum():
        acc_ref[...] += pacc
        cnt_ref[...] += pcnt

    @pl.when(i == NG - 1)
    def _finish():
        nf = acc_ref[...] / jnp.maximum(cnt_ref[...], 1.0)
        z = jnp.dot(nf, wc1h_ref[...], preferred_element_type=jnp.float32)
        z = z + jnp.dot(a2_ref[...], wc1a_ref[...],
                        preferred_element_type=jnp.float32) + bc1_ref[...]
        z = jnp.maximum(z, 0.0)
        z = jnp.maximum(jnp.dot(z, wc2_ref[...],
                                preferred_element_type=jnp.float32)
                        + bc2_ref[...], 0.0)
        q_ref[...] = jnp.dot(z, wc3_ref[...],
                             preferred_element_type=jnp.float32) + bc3_ref[...]


def _row_spec(cols):
    return pl.BlockSpec((RB, cols), lambda i: (i, 0))


def _full_spec(shape):
    nd = len(shape)
    return pl.BlockSpec(shape, lambda i, _nd=nd: (0,) * _nd)


def _part_spec():
    return pl.BlockSpec((RB, W128), lambda i: (i, 0))


_tc1 = pl.pallas_call(
    _tc1_body,
    grid=(NG,),
    in_specs=[_row_spec(D_IN), _full_spec((D_IN, W128))],
    out_specs=_row_spec(W128),
    out_shape=jax.ShapeDtypeStruct((N, W128), jnp.float32),
    compiler_params=pltpu.CompilerParams(
        dimension_semantics=("parallel",)),
)

_tc2 = pl.pallas_call(
    _tc2_body,
    grid=(NG,),
    in_specs=[_row_spec(D_IN), _part_spec(), _full_spec((D_IN, H)),
              _full_spec((1, H)),
              _full_spec((H, W128)), _full_spec((H, W128))],
    out_specs=(_row_spec(W128), _row_spec(W128)),
    out_shape=(jax.ShapeDtypeStruct((N, W128), jnp.float32),
               jax.ShapeDtypeStruct((N, W128), jnp.float32)),
    compiler_params=pltpu.CompilerParams(
        dimension_semantics=("parallel",)),
)

_tc3 = pl.pallas_call(
    _tc3_body,
    grid=(NG,),
    in_specs=[_row_spec(W128), _part_spec(), _full_spec((1, H)),
              _full_spec((H, W128)), _full_spec((H, W128))],
    out_specs=(_row_spec(W128), _row_spec(W128)),
    out_shape=(jax.ShapeDtypeStruct((N, W128), jnp.float32),
               jax.ShapeDtypeStruct((N, W128), jnp.float32)),
    compiler_params=pltpu.CompilerParams(
        dimension_semantics=("parallel",)),
)

_tc4 = pl.pallas_call(
    _tc4_body,
    grid=(NG,),
    in_specs=[_row_spec(W128), _part_spec(), _full_spec((1, H)),
              _row_spec(W128), _full_spec((B, A_DIM)),
              _full_spec((H, NET)), _full_spec((A_DIM, NET)),
              _full_spec((1, NET)), _full_spec((NET, NET)),
              _full_spec((1, NET)), _full_spec((NET, R_DIM)),
              _full_spec((1, R_DIM))],
    out_specs=_full_spec((B, R_DIM)),
    out_shape=jax.ShapeDtypeStruct((B, R_DIM), jnp.float32),
    scratch_shapes=[pltpu.VMEM((B, H), jnp.float32),
                    pltpu.VMEM((B, 1), jnp.float32)],
    compiler_params=pltpu.CompilerParams(
        dimension_semantics=("arbitrary",)),
)


def _pad128(w):
    return jnp.pad(w, ((0, 0), (0, W128 - w.shape[1])))


def kernel(x, edge_index, node_graph_ids, a,
           Wself0, Wneigh0, b0, Wself1, Wneigh1, b1, Wself2, Wneigh2, b2,
           Wc1, bc1, Wc2, bc2, Wc3, bc3):
    pad = EPAD - E
    src = jnp.concatenate([edge_index[0], jnp.zeros((pad,), jnp.int32)])
    dst = jnp.concatenate(
        [edge_index[1], (jnp.arange(pad, dtype=jnp.int32) % 16) + N])
    src = src.reshape(NW, NCHUNK, CH)
    dst = dst.reshape(NW, NCHUNK, CH)
    zeros = jnp.zeros((NP, H), jnp.float32)
    ones = jnp.ones((CH, DW), jnp.float32)

    g0 = _tc1(x, _pad128(Wneigh0))
    sd0 = _sc_scatter_deg(src, dst, g0, zeros, ones)
    h1s, g1 = _tc2(x, sd0, Wself0, b0.reshape(1, H),
                   _pad128(Wself1), _pad128(Wneigh1))
    s1 = _sc_scatter(src, dst, g1, zeros)
    h2s, g2 = _tc3(h1s, s1, b1.reshape(1, H),
                   _pad128(Wself2), _pad128(Wneigh2))
    s2 = _sc_scatter(src, dst, g2, zeros)

    mask = (node_graph_ids[:, None]
            == jnp.arange(W128, dtype=jnp.int32)[None, :]).astype(jnp.float32)
    a2 = jnp.squeeze(a, -1)
    q = _tc4(h2s, s2, b2.reshape(1, H), mask, a2,
             Wc1[:H], Wc1[H:], bc1.reshape(1, NET), Wc2,
             bc2.reshape(1, NET), Wc3, bc3.reshape(1, R_DIM))
    return q


# speedup vs baseline: 25.2162x; 1.0301x over previous
import functools

import jax
import jax.numpy as jnp
from jax import lax
from jax.experimental import pallas as pl
from jax.experimental.pallas import tpu as pltpu
from jax.experimental.pallas import tpu_sc as plsc

N = 10000
E = 320000
D_IN = 128
H = 32
B = 16
A_DIM = 8
R_DIM = 4
NET = 256
W128 = 128
DW = 8

NC = 2
NS = 16
NW = NC * NS
CH = 128
NCHUNK = 80
EPT = NCHUNK * CH
EPAD = NW * EPT
NP = 10112
RZ = NP // NS
NLAST = N - (NS - 1) * RZ

NBUF = 4

RB = 2000
NG = N // RB

_MESH = plsc.VectorSubcoreMesh(
    core_axis_name="c", subcore_axis_name="s", num_cores=NC, num_subcores=NS)


def _sc_edge_body(with_deg, src_hbm, dst_hbm, g_hbm, zeros_hbm, *rest):
    if with_deg:
        (ones_hbm, out_hbm,
         acc_sp, deg_sp, g_sp, src_v, dst_v, rows_v, ones_v, sems, dsem) = rest
    else:
        (out_hbm, acc_sp, g_sp, src_v, dst_v, rows_v, sems) = rest
    c = lax.axis_index("c")
    s = lax.axis_index("s")
    wid = c * NS + s

    gsems, ssems = sems
    psem = gsems.at[NBUF - 1]
    pro = [
        pltpu.async_copy(zeros_hbm.at[pl.ds(s * RZ, RZ)],
                         acc_sp.at[pl.ds(s * RZ, RZ)], psem),
        pltpu.async_copy(src_hbm.at[wid], src_v, psem),
        pltpu.async_copy(dst_hbm.at[wid], dst_v, psem),
    ]

    @pl.when(s < NS - 1)
    def _stage():
        pltpu.async_copy(g_hbm.at[pl.ds(s * RZ, RZ), pl.ds(0, H)],
                         g_sp.at[pl.ds(s * RZ, RZ)], psem)

    @pl.when(s == NS - 1)
    def _stage_last():
        pltpu.async_copy(g_hbm.at[pl.ds((NS - 1) * RZ, NLAST), pl.ds(0, H)],
                         g_sp.at[pl.ds((NS - 1) * RZ, NLAST)], psem)

    if with_deg:
        pro.append(pltpu.async_copy(zeros_hbm.at[pl.ds(s * RZ, RZ), pl.ds(0, DW)],
                                    deg_sp.at[pl.ds(s * RZ, RZ)], psem))
        pro.append(pltpu.async_copy(ones_hbm, ones_v, psem))
    for d in pro:
        d.wait()

    @pl.when(s < NS - 1)
    def _stage_wait():
        pltpu.make_async_copy(g_hbm.at[pl.ds(s * RZ, RZ), pl.ds(0, H)],
                              g_sp.at[pl.ds(s * RZ, RZ)], psem).wait()

    @pl.when(s == NS - 1)
    def _stage_wait_last():
        pltpu.make_async_copy(
            g_hbm.at[pl.ds((NS - 1) * RZ, NLAST), pl.ds(0, H)],
            g_sp.at[pl.ds((NS - 1) * RZ, NLAST)], psem).wait()
    plsc.subcore_barrier()

    pltpu.async_copy(g_sp.at[src_v.at[0]], rows_v.at[0], gsems.at[0])
    pltpu.async_copy(g_sp.at[src_v.at[1]], rows_v.at[1], gsems.at[1])

    def outer(jj, carry):
        for b in range(NBUF):
            i = jj * NBUF + b
            bn = (b + 2) % NBUF

            @pl.when(jnp.logical_and(i >= 2, i + 2 < NCHUNK))
            def _drain():
                pltpu.make_async_copy(
                    rows_v.at[bn], acc_sp.at[dst_v.at[i]], ssems.at[bn]).wait()

            @pl.when(i + 2 < NCHUNK)
            def _fire():
                pltpu.async_copy(
                    g_sp.at[src_v.at[i + 2]], rows_v.at[bn], gsems.at[bn])

            pltpu.make_async_copy(
                g_sp.at[src_v.at[i]], rows_v.at[b], gsems.at[b]).wait()
            pltpu.async_copy(rows_v.at[b], acc_sp.at[dst_v.at[i]],
                             ssems.at[b], add=True)
            if with_deg:
                pltpu.async_copy(ones_v, deg_sp.at[dst_v.at[i]],
                                 dsem, add=True)
        return carry

    lax.fori_loop(0, NCHUNK // NBUF, outer, 0)

    for b in range(NBUF):
        pltpu.make_async_copy(
            rows_v.at[b], acc_sp.at[dst_v.at[0]], ssems.at[b]).wait()
    if with_deg:
        def drain_deg(i, carry):
            pltpu.make_async_copy(
                ones_v, deg_sp.at[dst_v.at[0]], dsem).wait()
            return carry
        lax.fori_loop(0, NCHUNK, drain_deg, 0)
    plsc.subcore_barrier()

    for cc in range(NC):
        @pl.when(c == cc)
        def _copy_out(cc=cc):
            pltpu.sync_copy(acc_sp.at[pl.ds(s * RZ, RZ)],
                            out_hbm.at[pl.ds(s * RZ, RZ), pl.ds(cc * 64, H)])
            if with_deg:
                pltpu.sync_copy(
                    deg_sp.at[pl.ds(s * RZ, RZ)],
                    out_hbm.at[pl.ds(s * RZ, RZ), pl.ds(cc * 64 + H, DW)])


_sc_scatter_deg = pl.kernel(
    functools.partial(_sc_edge_body, True),
    out_type=jax.ShapeDtypeStruct((NP, W128), jnp.float32),
    mesh=_MESH,
    compiler_params=pltpu.CompilerParams(use_tc_tiling_on_sc=False),
    scratch_types=[
        pltpu.VMEM_SHARED((NP, H), jnp.float32),
        pltpu.VMEM_SHARED((NP, DW), jnp.float32),
        pltpu.VMEM_SHARED((N, H), jnp.float32),
        pltpu.VMEM((NCHUNK, CH), jnp.int32),
        pltpu.VMEM((NCHUNK, CH), jnp.int32),
        pltpu.VMEM((NBUF, CH, H), jnp.float32),
        pltpu.VMEM((CH, DW), jnp.float32),
        (pltpu.SemaphoreType.DMA((NBUF,)), pltpu.SemaphoreType.DMA((NBUF,))),
        pltpu.SemaphoreType.DMA,
    ],
)

_sc_scatter = pl.kernel(
    functools.partial(_sc_edge_body, False),
    out_type=jax.ShapeDtypeStruct((NP, W128), jnp.float32),
    mesh=_MESH,
    compiler_params=pltpu.CompilerParams(use_tc_tiling_on_sc=False),
    scratch_types=[
        pltpu.VMEM_SHARED((NP, H), jnp.float32),
        pltpu.VMEM_SHARED((N, H), jnp.float32),
        pltpu.VMEM((NCHUNK, CH), jnp.int32),
        pltpu.VMEM((NCHUNK, CH), jnp.int32),
        pltpu.VMEM((NBUF, CH, H), jnp.float32),
        (pltpu.SemaphoreType.DMA((NBUF,)), pltpu.SemaphoreType.DMA((NBUF,))),
    ],
)


def _col32():
    return (lax.broadcasted_iota(jnp.int32, (1, W128), 1) == H).astype(
        jnp.float32)


def _tc1_body(x_ref, wn_ref, g_ref):
    g_ref[...] = jnp.dot(x_ref[...], wn_ref[...],
                         preferred_element_type=jnp.float32)


def _tc2_body(x_ref, sd_ref, ws0_ref, b_ref, wsn_ref, wnn_ref, hs_ref, g_ref):
    ssum = sd_ref[:, :H] + sd_ref[:, 64:64 + H]
    deg = sd_ref[:, H:H + 1] + sd_ref[:, 96:97]
    dinv = 1.0 / jnp.maximum(deg, 1.0)
    xs = jnp.dot(x_ref[...], ws0_ref[...], preferred_element_type=jnp.float32)
    h = jnp.maximum(xs + ssum * dinv + b_ref[...], 0.0)
    hs_ref[...] = (jnp.dot(h, wsn_ref[...], preferred_element_type=jnp.float32)
                   + dinv * _col32())
    g_ref[...] = jnp.dot(h, wnn_ref[...], preferred_element_type=jnp.float32)


def _tc3_body(hs_ref, sp_ref, b_ref, wsn_ref, wnn_ref, hs2_ref, g2_ref):
    ssum = sp_ref[:, :H] + sp_ref[:, 64:64 + H]
    dinv = hs_ref[:, H:H + 1]
    h = jnp.maximum(hs_ref[:, :H] + ssum * dinv + b_ref[...], 0.0)
    hs2_ref[...] = (jnp.dot(h, wsn_ref[...], preferred_element_type=jnp.float32)
                    + dinv * _col32())
    g2_ref[...] = jnp.dot(h, wnn_ref[...], preferred_element_type=jnp.float32)


def _tc4_body(hs_ref, sp_ref, b_ref, mask_ref, a2_ref,
              wc1h_ref, wc1a_ref, bc1_ref, wc2_ref, bc2_ref, wc3_ref, bc3_ref,
              q_ref, acc_ref, cnt_ref):
    i = pl.program_id(0)
    ssum = sp_ref[:, :H] + sp_ref[:, 64:64 + H]
    dinv = hs_ref[:, H:H + 1]
    h3 = jnp.maximum(hs_ref[:, :H] + ssum * dinv + b_ref[...], 0.0)
    m = mask_ref[:, :B]
    dn = (((0,), (0,)), ((), ()))
    pacc = lax.dot_general(m, h3, dn,
                           preferred_element_type=jnp.float32)
    pcnt = lax.dot_general(m, jnp.ones((RB, 1), jnp.float32), dn,
                           preferred_element_type=jnp.float32)

    @pl.when(i == 0)
    def _init():
        acc_ref[...] = pacc
        cnt_ref[...] = pcnt

    @pl.when(i > 0)
    def _accum():
        acc_ref[...] += pacc
        cnt_ref[...] += pcnt

    @pl.when(i == NG - 1)
    def _finish():
        nf = acc_ref[...] / jnp.maximum(cnt_ref[...], 1.0)
        z = jnp.dot(nf, wc1h_ref[...], preferred_element_type=jnp.float32)
        z = z + jnp.dot(a2_ref[...], wc1a_ref[...],
                        preferred_element_type=jnp.float32) + bc1_ref[...]
        z = jnp.maximum(z, 0.0)
        z = jnp.maximum(jnp.dot(z, wc2_ref[...],
                                preferred_element_type=jnp.float32)
                        + bc2_ref[...], 0.0)
        q_ref[...] = jnp.dot(z, wc3_ref[...],
                             preferred_element_type=jnp.float32) + bc3_ref[...]


def _row_spec(cols):
    return pl.BlockSpec((RB, cols), lambda i: (i, 0))


def _full_spec(shape):
    nd = len(shape)
    return pl.BlockSpec(shape, lambda i, _nd=nd: (0,) * _nd)


def _part_spec():
    return pl.BlockSpec((RB, W128), lambda i: (i, 0))


_tc1 = pl.pallas_call(
    _tc1_body,
    grid=(NG,),
    in_specs=[_row_spec(D_IN), _full_spec((D_IN, W128))],
    out_specs=_row_spec(W128),
    out_shape=jax.ShapeDtypeStruct((N, W128), jnp.float32),
    compiler_params=pltpu.CompilerParams(
        dimension_semantics=("parallel",)),
)

_tc2 = pl.pallas_call(
    _tc2_body,
    grid=(NG,),
    in_specs=[_row_spec(D_IN), _part_spec(), _full_spec((D_IN, H)),
              _full_spec((1, H)),
              _full_spec((H, W128)), _full_spec((H, W128))],
    out_specs=(_row_spec(W128), _row_spec(W128)),
    out_shape=(jax.ShapeDtypeStruct((N, W128), jnp.float32),
               jax.ShapeDtypeStruct((N, W128), jnp.float32)),
    compiler_params=pltpu.CompilerParams(
        dimension_semantics=("parallel",)),
)

_tc3 = pl.pallas_call(
    _tc3_body,
    grid=(NG,),
    in_specs=[_row_spec(W128), _part_spec(), _full_spec((1, H)),
              _full_spec((H, W128)), _full_spec((H, W128))],
    out_specs=(_row_spec(W128), _row_spec(W128)),
    out_shape=(jax.ShapeDtypeStruct((N, W128), jnp.float32),
               jax.ShapeDtypeStruct((N, W128), jnp.float32)),
    compiler_params=pltpu.CompilerParams(
        dimension_semantics=("parallel",)),
)

_tc4 = pl.pallas_call(
    _tc4_body,
    grid=(NG,),
    in_specs=[_row_spec(W128), _part_spec(), _full_spec((1, H)),
              _row_spec(W128), _full_spec((B, A_DIM)),
              _full_spec((H, NET)), _full_spec((A_DIM, NET)),
              _full_spec((1, NET)), _full_spec((NET, NET)),
              _full_spec((1, NET)), _full_spec((NET, R_DIM)),
              _full_spec((1, R_DIM))],
    out_specs=_full_spec((B, R_DIM)),
    out_shape=jax.ShapeDtypeStruct((B, R_DIM), jnp.float32),
    scratch_shapes=[pltpu.VMEM((B, H), jnp.float32),
                    pltpu.VMEM((B, 1), jnp.float32)],
    compiler_params=pltpu.CompilerParams(
        dimension_semantics=("arbitrary",)),
)


def _pad128(w):
    return jnp.pad(w, ((0, 0), (0, W128 - w.shape[1])))


def kernel(x, edge_index, node_graph_ids, a,
           Wself0, Wneigh0, b0, Wself1, Wneigh1, b1, Wself2, Wneigh2, b2,
           Wc1, bc1, Wc2, bc2, Wc3, bc3):
    pad = EPAD - E
    src = jnp.concatenate([edge_index[0], jnp.zeros((pad,), jnp.int32)])
    dst = jnp.concatenate(
        [edge_index[1], (jnp.arange(pad, dtype=jnp.int32) % 16) + N])
    src = src.reshape(NW, NCHUNK, CH)
    dst = dst.reshape(NW, NCHUNK, CH)
    zeros = jnp.zeros((NP, H), jnp.float32)
    ones = jnp.ones((CH, DW), jnp.float32)

    g0 = _tc1(x, _pad128(Wneigh0))
    sd0 = _sc_scatter_deg(src, dst, g0, zeros, ones)
    h1s, g1 = _tc2(x, sd0, Wself0, b0.reshape(1, H),
                   _pad128(Wself1), _pad128(Wneigh1))
    s1 = _sc_scatter(src, dst, g1, zeros)
    h2s, g2 = _tc3(h1s, s1, b1.reshape(1, H),
                   _pad128(Wself2), _pad128(Wneigh2))
    s2 = _sc_scatter(src, dst, g2, zeros)

    mask = (node_graph_ids[:, None]
            == jnp.arange(W128, dtype=jnp.int32)[None, :]).astype(jnp.float32)
    a2 = jnp.squeeze(a, -1)
    q = _tc4(h2s, s2, b2.reshape(1, H), mask, a2,
             Wc1[:H], Wc1[H:], bc1.reshape(1, NET), Wc2,
             bc2.reshape(1, NET), Wc3, bc3.reshape(1, R_DIM))
    return q
